# Initial kernel scaffold; baseline (speedup 1.0000x reference)
import functools
import jax
import jax.numpy as jnp
from jax import lax
from jax.experimental import pallas as pl
from jax.experimental.pallas import tpu as pltpu
from jax.experimental.pallas import tpu_sc as plsc

NC, NS, L = 2, 16, 16
NW = NC * NS
_MESH = plsc.VectorSubcoreMesh(core_axis_name="c", subcore_axis_name="s")


@functools.partial(
    pl.kernel,
    out_type=jax.ShapeDtypeStruct((NW, 32), jnp.float32),
    mesh=_MESH,
    scratch_types=[pltpu.VMEM((32,), jnp.float32)],
)
def _probe(out_hbm, hist):
    wid = lax.axis_index("s") * NC + lax.axis_index("c")
    zero = jnp.zeros((16,), jnp.float32)
    hist[pl.ds(0, 16)] = zero
    hist[pl.ds(16, 16)] = zero
    idx = lax.iota(jnp.int32, 16) // 4
    ones = jnp.ones((16,), jnp.float32)
    plsc.addupdate_scatter(hist, [idx], ones)
    idx2 = jnp.minimum(lax.iota(jnp.int32, 16), 2) + 8
    plsc.addupdate_scatter(hist, [idx2], ones)
    pltpu.sync_copy(hist, out_hbm.at[wid])


def kernel(x, res_size1, edge_index1, edge_attr1, res_size2, edge_index2, edge_attr2,
           W_nbr1, W_self1, b1, gamma, beta, W_nbr2, W_self2, b2):
    r = _probe()
    return jnp.zeros((1000, 128), jnp.float32) + r[0, 0]


# speedup vs baseline: 3.2112x; 3.2112x over previous
import functools

import jax
import jax.numpy as jnp
from jax import lax
from jax.experimental import pallas as pl
from jax.experimental.pallas import tpu as pltpu
from jax.experimental.pallas import tpu_sc as plsc

NC, NS, LANES = 2, 16, 16
NW = NC * NS

N1, N2 = 2500, 1000
E1, E2 = 160000, 40000
G1, G2 = E1 // 16, E2 // 16
DE = 16

R1_SLOT = 40
R2_SLOT = 32
A1_ROWS, A2_ROWS = 2560, 1024
BLK1 = R1_SLOT * N1
BLK2 = R2_SLOT * N2
CHUNK = 8000
SE1_GCAP = 157
SE2_GCAP = 79
SE1_ROWS = 2560
SE1_W = SE1_ROWS * DE
SE2_W = N2 * DE
EA_OFF = 41024

_MESH = plsc.VectorSubcoreMesh(core_axis_name="c", subcore_axis_name="s")


@functools.partial(
    pl.kernel,
    out_type=(
        jax.ShapeDtypeStruct((A1_ROWS * N1,), jnp.float32),
        jax.ShapeDtypeStruct((A2_ROWS * N2,), jnp.float32),
        jax.ShapeDtypeStruct((NW * SE1_W,), jnp.float32),
        jax.ShapeDtypeStruct((NW * SE2_W,), jnp.float32),
    ),
    mesh=_MESH,
    scratch_types=[
        pltpu.VMEM((102400,), jnp.float32),
        pltpu.VMEM((CHUNK,), jnp.int32),
        pltpu.VMEM((CHUNK,), jnp.int32),
    ],
    compiler_params=pltpu.CompilerParams(needs_layout_passes=False),
)
def _sc_build(src1, dst1, ea1, src2, dst2, ea2, a1_out, a2_out, se1_out, se2_out,
              buf, idx_a, idx_b):
    wid = lax.axis_index("s") * NC + lax.axis_index("c")
    zeros16 = jnp.zeros((16,), jnp.float32)
    ones16 = jnp.ones((16,), jnp.float32)
    lane = lax.iota(jnp.int32, 16)

    def zero_buf(nwords):
        def body(i, _):
            buf[pl.ds(i * 16, 16)] = zeros16
            return 0
        lax.fori_loop(0, nwords // 16, body, 0)

    def adj_phase(src, dst, n_edges, ncols, blk, slot, out_ref):
        zero_buf(blk)
        lo = slot * blk
        nchunks = n_edges // CHUNK

        def chunk_body(c, _):
            pltpu.sync_copy(src.at[pl.ds(c * CHUNK, CHUNK)], idx_a)
            pltpu.sync_copy(dst.at[pl.ds(c * CHUNK, CHUNK)], idx_b)

            def vec_body(k, _):
                s = idx_a[pl.ds(k * 16, 16)]
                d = idx_b[pl.ds(k * 16, 16)]
                flat = d * ncols + s - lo
                valid = (flat >= 0) & (flat < blk)
                plsc.addupdate_scatter(buf.at[pl.ds(0, blk)], [flat], ones16,
                                       mask=valid)
                return 0
            lax.fori_loop(0, CHUNK // 16, vec_body, 0)
            return 0
        lax.fori_loop(0, nchunks, chunk_body, 0)
        pltpu.sync_copy(buf.at[pl.ds(0, blk)], out_ref.at[pl.ds(slot * blk, blk)])

    for p in range(2):
        adj_phase(src1, dst1, E1, N1, BLK1, wid + p * NW, a1_out)
    adj_phase(src2, dst2, E2, N2, BLK2, wid, a2_out)

    def se_phase(dst, ea, groups, gcap, nchunk, priv_w, out_ref):
        g0 = (wid * groups) // NW
        g1 = ((wid + 1) * groups) // NW
        ng = g1 - g0
        zero_buf(priv_w)
        priv = buf.at[pl.ds(0, priv_w)]
        ea_stage = buf.at[pl.ds(EA_OFF, gcap * 256)]
        for c in range(nchunk):
            gbase = g0 + c * gcap
            pltpu.sync_copy(dst.at[pl.ds(gbase * 16, gcap * 16)],
                            idx_b.at[pl.ds(0, gcap * 16)])
            pltpu.sync_copy(ea.at[pl.ds(gbase * 256, gcap * 256)], ea_stage)

            def g_body(k, _):
                d = idx_b[pl.ds(k * 16, 16)]
                rowbase = d * DE
                valid = jnp.broadcast_to(c * gcap + k < ng, (16,))
                ebase = k * 256 + lane * 16
                for cc in range(DE):
                    vals = plsc.load_gather(ea_stage, [ebase + cc])
                    plsc.addupdate_scatter(priv, [rowbase + cc], vals,
                                           mask=valid)
                return 0
            lax.fori_loop(0, gcap, g_body, 0)
        pltpu.sync_copy(priv, out_ref.at[pl.ds(wid * priv_w, priv_w)])

    se_phase(dst1, ea1, G1, SE1_GCAP, 2, SE1_W, se1_out)
    se_phase(dst2, ea2, G2, SE2_GCAP, 1, SE2_W, se2_out)


_TC1_BLK = 512
_TC1_GRID = A1_ROWS // _TC1_BLK


def _tc1_body(a1, x_full, x_blk, se1p, wn1x, wn1e, wself, b1, gsc, beta, h_out):
    A = a1[...]
    S = jnp.dot(A, x_full[...], preferred_element_type=jnp.float32)
    cnt = jnp.sum(A, axis=1, keepdims=True)
    inv = 1.0 / jnp.maximum(cnt, 1.0)
    se = jnp.sum(se1p[...], axis=0)
    t = jnp.dot(S * inv, wn1x[...], preferred_element_type=jnp.float32)
    t = t + jnp.dot(se * inv, wn1e[...], preferred_element_type=jnp.float32)
    t = t + jnp.dot(x_blk[...], wself[...], preferred_element_type=jnp.float32)
    t = t + b1[...]
    h_out[...] = jnp.maximum(t, 0.0) * gsc[...] + beta[...]


_tc1 = pl.pallas_call(
    _tc1_body,
    grid=(_TC1_GRID,),
    in_specs=[
        pl.BlockSpec((_TC1_BLK, N1), lambda i: (i, 0)),
        pl.BlockSpec((N1, 256), lambda i: (0, 0)),
        pl.BlockSpec((_TC1_BLK, 256), lambda i: (i, 0)),
        pl.BlockSpec((NW, _TC1_BLK, DE), lambda i: (0, i, 0)),
        pl.BlockSpec((256, 1500), lambda i: (0, 0)),
        pl.BlockSpec((DE, 1500), lambda i: (0, 0)),
        pl.BlockSpec((256, 1500), lambda i: (0, 0)),
        pl.BlockSpec((1, 1500), lambda i: (0, 0)),
        pl.BlockSpec((1, 1500), lambda i: (0, 0)),
        pl.BlockSpec((1, 1500), lambda i: (0, 0)),
    ],
    out_specs=pl.BlockSpec((_TC1_BLK, 1500), lambda i: (i, 0)),
    out_shape=jax.ShapeDtypeStruct((A1_ROWS, 1500), jnp.float32),
    compiler_params=pltpu.CompilerParams(dimension_semantics=("parallel",)),
)


def _tc2_body(a2, h1k, se2p, wn2h, wn2e, wself2, b2, out):
    A = a2[...]
    H = h1k[...]
    P = jnp.dot(H, wn2h[...], preferred_element_type=jnp.float32)
    G = jnp.dot(A, P, preferred_element_type=jnp.float32)
    cnt = jnp.sum(A, axis=1, keepdims=True)
    inv = 1.0 / jnp.maximum(cnt, 1.0)
    se = jnp.sum(se2p[...], axis=0)
    o = (G * inv
         + jnp.dot(se * inv, wn2e[...], preferred_element_type=jnp.float32)
         + jnp.dot(H, wself2[...], preferred_element_type=jnp.float32)
         + b2[...])
    m = jnp.max(o, axis=1, keepdims=True)
    e = jnp.exp(o - m)
    out[...] = (o - m) - jnp.log(jnp.sum(e, axis=1, keepdims=True))


_tc2 = pl.pallas_call(
    _tc2_body,
    out_shape=jax.ShapeDtypeStruct((N2, 128), jnp.float32),
)


def kernel(x, res_size1, edge_index1, edge_attr1, res_size2, edge_index2,
           edge_attr2, W_nbr1, W_self1, b1, gamma, beta, W_nbr2, W_self2, b2):
    src1 = edge_index1[0].astype(jnp.int32)
    dst1 = jnp.pad(edge_index1[1].astype(jnp.int32), (0, 512))
    src2 = edge_index2[0].astype(jnp.int32)
    dst2 = edge_index2[1].astype(jnp.int32)
    ea1 = jnp.pad(edge_attr1.reshape(-1), (0, 512 * DE))
    ea2 = edge_attr2.reshape(-1)

    a1f, a2f, se1f, se2f = _sc_build(src1, dst1, ea1, src2, dst2, ea2)
    A1 = a1f.reshape(A1_ROWS, N1)
    A2 = a2f.reshape(A2_ROWS, N2)[:N2]
    SE1 = se1f.reshape(NW, SE1_ROWS, DE)
    SE2 = se2f.reshape(NW, N2, DE)

    x25 = jnp.pad(x[:N1], ((0, A1_ROWS - N1), (0, 0)))
    gscale = (gamma * (1.0 / jnp.sqrt(jnp.float32(1.0 + 1e-5))))[None]

    h = _tc1(A1, x25[:N1], x25, SE1, W_nbr1[:256], W_nbr1[256:], W_self1,
             b1[None], gscale, beta[None])
    out = _tc2(A2, h[:N2], SE2, W_nbr2[:1500], W_nbr2[1500:], W_self2,
               b2[None])
    return out


# speedup vs baseline: 4.2109x; 1.3113x over previous
import functools

import jax
import jax.numpy as jnp
from jax import lax
from jax.experimental import pallas as pl
from jax.experimental.pallas import tpu as pltpu
from jax.experimental.pallas import tpu_sc as plsc

NC, NS, LANES = 2, 16, 16
NW = NC * NS

N1, N2 = 2500, 1000
E1, E2 = 160000, 40000
G1, G2 = E1 // 16, E2 // 16
DE = 16

R1_SLOT = 40
R2_SLOT = 32
A1_ROWS, A2_ROWS = 2560, 1024
BLK1 = R1_SLOT * N1
BLK2 = R2_SLOT * N2
CHUNK = 6400
NCH1 = E1 // CHUNK
E2_PAD = 44800
NCH2 = E2_PAD // CHUNK
DST_PAD_VAL = 800000
SE1_GCAP = 157
SE2_GCAP = 79
SE1_ROWS = 2560
SE1_W = SE1_ROWS * DE
SE2_W = N2 * DE
EA_OFF = 41024

_MESH = plsc.VectorSubcoreMesh(core_axis_name="c", subcore_axis_name="s")


@functools.partial(
    pl.kernel,
    out_type=(
        jax.ShapeDtypeStruct((A1_ROWS * N1,), jnp.float32),
        jax.ShapeDtypeStruct((A2_ROWS * N2,), jnp.float32),
        jax.ShapeDtypeStruct((NW * SE1_W,), jnp.float32),
        jax.ShapeDtypeStruct((NW * SE2_W,), jnp.float32),
    ),
    mesh=_MESH,
    scratch_types=[
        pltpu.VMEM((102400,), jnp.float32),
        pltpu.VMEM((2 * CHUNK,), jnp.int32),
        pltpu.VMEM((2 * CHUNK,), jnp.int32),
        pltpu.SemaphoreType.DMA,
        pltpu.SemaphoreType.DMA,
    ],
    compiler_params=pltpu.CompilerParams(needs_layout_passes=False),
)
def _sc_build(src1, dst1, ea1, src2, dst2, ea2, a1_out, a2_out, se1_out, se2_out,
              buf, idx_s, idx_d, sem_a, sem_b):
    wid = lax.axis_index("s") * NC + lax.axis_index("c")
    zeros16 = jnp.zeros((16,), jnp.float32)
    ones16 = jnp.ones((16,), jnp.float32)
    lane = lax.iota(jnp.int32, 16)

    def zero_buf(nwords):
        def body(i, _):
            for u in range(10):
                buf[pl.ds(i * 160 + u * 16, 16)] = zeros16
            return 0
        lax.fori_loop(0, nwords // 160, body, 0)

    def adj_phase(src, dst, nchunks, ncols, blk, npasses, out_ref):
        ublk = jnp.uint32(blk)

        def scan(base, lo):
            def vec_body(k, _):
                kb = base + k * 64
                for u in range(4):
                    s = idx_s[pl.ds(kb + u * 16, 16)]
                    d = idx_d[pl.ds(kb + u * 16, 16)]
                    flat = d * ncols + s - lo
                    valid = plsc.bitcast(flat, jnp.uint32) < ublk
                    plsc.addupdate_scatter(buf.at[pl.ds(0, blk)], [flat],
                                           ones16, mask=valid)
                return 0
            lax.fori_loop(0, CHUNK // 64, vec_body, 0)

        def issue(c, half):
            sem = sem_a if half == 0 else sem_b
            pltpu.async_copy(src.at[pl.ds(c * CHUNK, CHUNK)],
                             idx_s.at[pl.ds(half * CHUNK, CHUNK)], sem)
            pltpu.async_copy(dst.at[pl.ds(c * CHUNK, CHUNK)],
                             idx_d.at[pl.ds(half * CHUNK, CHUNK)], sem)

        def drain(c, half):
            sem = sem_a if half == 0 else sem_b
            pltpu.make_async_copy(src.at[pl.ds(c * CHUNK, CHUNK)],
                                  idx_s.at[pl.ds(half * CHUNK, CHUNK)],
                                  sem).wait()
            pltpu.make_async_copy(dst.at[pl.ds(c * CHUNK, CHUNK)],
                                  idx_d.at[pl.ds(half * CHUNK, CHUNK)],
                                  sem).wait()

        def pass_body(p, _):
            slot = wid + p * NW
            lo = slot * blk
            issue(0, 0)
            zero_buf(blk)

            def chunk_body(c, _):
                nxt = c + 1

                @pl.when(nxt < nchunks)
                def _():
                    @pl.when(c % 2 == 0)
                    def _():
                        issue(nxt, 1)

                    @pl.when(c % 2 == 1)
                    def _():
                        issue(nxt, 0)

                @pl.when(c % 2 == 0)
                def _():
                    drain(c, 0)
                    scan(0, lo)

                @pl.when(c % 2 == 1)
                def _():
                    drain(c, 1)
                    scan(CHUNK, lo)

                return 0
            lax.fori_loop(0, nchunks, chunk_body, 0)
            pltpu.sync_copy(buf.at[pl.ds(0, blk)],
                            out_ref.at[pl.ds(slot * blk, blk)])
            return 0
        lax.fori_loop(0, npasses, pass_body, 0)

    adj_phase(src1, dst1, NCH1, N1, BLK1, 2, a1_out)
    adj_phase(src2, dst2, NCH2, N2, BLK2, 1, a2_out)

    def se_phase(dst, ea, groups, gcap, nchunk, priv_w, out_ref):
        g0 = (wid * groups) // NW
        g1 = ((wid + 1) * groups) // NW
        ng = g1 - g0
        zero_buf(priv_w)
        priv = buf.at[pl.ds(0, priv_w)]
        ea_stage = buf.at[pl.ds(EA_OFF, gcap * 256)]
        for c in range(nchunk):
            gbase = g0 + c * gcap
            pltpu.sync_copy(dst.at[pl.ds(gbase * 16, gcap * 16)],
                            idx_d.at[pl.ds(0, gcap * 16)])
            pltpu.sync_copy(ea.at[pl.ds(gbase * 256, gcap * 256)], ea_stage)

            def g_body(k, _):
                d = idx_d[pl.ds(k * 16, 16)]
                rowbase = d * DE
                valid = jnp.broadcast_to(c * gcap + k < ng, (16,))
                ebase = k * 256 + lane * 16
                for cc in range(DE):
                    vals = plsc.load_gather(ea_stage, [ebase + cc])
                    plsc.addupdate_scatter(priv, [rowbase + cc], vals,
                                           mask=valid)
                return 0
            lax.fori_loop(0, gcap, g_body, 0)
        pltpu.sync_copy(priv, out_ref.at[pl.ds(wid * priv_w, priv_w)])

    se_phase(dst1, ea1, G1, SE1_GCAP, 2, SE1_W, se1_out)
    se_phase(dst2, ea2, G2, SE2_GCAP, 1, SE2_W, se2_out)


_TC1_BLK = 512
_TC1_GRID = A1_ROWS // _TC1_BLK


def _tc1_body(a1, x_full, x_blk, se1p, wn1x, wn1e, wself, b1, gsc, beta, h_out):
    A = a1[...]
    S = jnp.dot(A, x_full[...], preferred_element_type=jnp.float32)
    cnt = jnp.sum(A, axis=1, keepdims=True)
    inv = 1.0 / jnp.maximum(cnt, 1.0)
    se = jnp.sum(se1p[...], axis=0)
    t = jnp.dot(S * inv, wn1x[...], preferred_element_type=jnp.float32)
    t = t + jnp.dot(se * inv, wn1e[...], preferred_element_type=jnp.float32)
    t = t + jnp.dot(x_blk[...], wself[...], preferred_element_type=jnp.float32)
    t = t + b1[...]
    h_out[...] = jnp.maximum(t, 0.0) * gsc[...] + beta[...]


_tc1 = pl.pallas_call(
    _tc1_body,
    grid=(_TC1_GRID,),
    in_specs=[
        pl.BlockSpec((_TC1_BLK, N1), lambda i: (i, 0)),
        pl.BlockSpec((N1, 256), lambda i: (0, 0)),
        pl.BlockSpec((_TC1_BLK, 256), lambda i: (i, 0)),
        pl.BlockSpec((NW, _TC1_BLK, DE), lambda i: (0, i, 0)),
        pl.BlockSpec((256, 1500), lambda i: (0, 0)),
        pl.BlockSpec((DE, 1500), lambda i: (0, 0)),
        pl.BlockSpec((256, 1500), lambda i: (0, 0)),
        pl.BlockSpec((1, 1500), lambda i: (0, 0)),
        pl.BlockSpec((1, 1500), lambda i: (0, 0)),
        pl.BlockSpec((1, 1500), lambda i: (0, 0)),
    ],
    out_specs=pl.BlockSpec((_TC1_BLK, 1500), lambda i: (i, 0)),
    out_shape=jax.ShapeDtypeStruct((A1_ROWS, 1500), jnp.float32),
    compiler_params=pltpu.CompilerParams(dimension_semantics=("parallel",)),
)


def _tc2_body(a2, h1k, se2p, wn2h, wn2e, wself2, b2, out):
    A = a2[...]
    H = h1k[...]
    P = jnp.dot(H, wn2h[...], preferred_element_type=jnp.float32)
    G = jnp.dot(A, P, preferred_element_type=jnp.float32)
    cnt = jnp.sum(A, axis=1, keepdims=True)
    inv = 1.0 / jnp.maximum(cnt, 1.0)
    se = jnp.sum(se2p[...], axis=0)
    o = (G * inv
         + jnp.dot(se * inv, wn2e[...], preferred_element_type=jnp.float32)
         + jnp.dot(H, wself2[...], preferred_element_type=jnp.float32)
         + b2[...])
    m = jnp.max(o, axis=1, keepdims=True)
    e = jnp.exp(o - m)
    out[...] = (o - m) - jnp.log(jnp.sum(e, axis=1, keepdims=True))


_tc2 = pl.pallas_call(
    _tc2_body,
    out_shape=jax.ShapeDtypeStruct((N2, 128), jnp.float32),
)


def kernel(x, res_size1, edge_index1, edge_attr1, res_size2, edge_index2,
           edge_attr2, W_nbr1, W_self1, b1, gamma, beta, W_nbr2, W_self2, b2):
    src1 = jnp.pad(edge_index1[0].astype(jnp.int32), (0, 6400))
    dst1 = jnp.pad(edge_index1[1].astype(jnp.int32), (0, 6400),
                   constant_values=DST_PAD_VAL)
    src2 = jnp.pad(edge_index2[0].astype(jnp.int32), (0, E2_PAD - E2))
    dst2 = jnp.pad(edge_index2[1].astype(jnp.int32), (0, E2_PAD - E2),
                   constant_values=DST_PAD_VAL)
    ea1 = jnp.pad(edge_attr1.reshape(-1), (0, 512 * DE))
    ea2 = edge_attr2.reshape(-1)

    a1f, a2f, se1f, se2f = _sc_build(src1, dst1, ea1, src2, dst2, ea2)
    A1 = a1f.reshape(A1_ROWS, N1)
    A2 = a2f.reshape(A2_ROWS, N2)[:N2]
    SE1 = se1f.reshape(NW, SE1_ROWS, DE)
    SE2 = se2f.reshape(NW, N2, DE)

    x25 = jnp.pad(x[:N1], ((0, A1_ROWS - N1), (0, 0)))
    gscale = (gamma * (1.0 / jnp.sqrt(jnp.float32(1.0 + 1e-5))))[None]

    h = _tc1(A1, x25[:N1], x25, SE1, W_nbr1[:256], W_nbr1[256:], W_self1,
             b1[None], gscale, beta[None])
    out = _tc2(A2, h[:N2], SE2, W_nbr2[:1500], W_nbr2[1500:], W_self2,
               b2[None])
    return out


# speedup vs baseline: 4.2175x; 1.0015x over previous
import functools

import jax
import jax.numpy as jnp
from jax import lax
from jax.experimental import pallas as pl
from jax.experimental.pallas import tpu as pltpu
from jax.experimental.pallas import tpu_sc as plsc

NC, NS, LANES = 2, 16, 16
NW = NC * NS

N1, N2 = 2500, 1000
E1, E2 = 160000, 40000
G1, G2 = E1 // 16, E2 // 16
DE = 16

R1_SLOT = 40
R2_SLOT = 32
A1_ROWS, A2_ROWS = 2560, 1024
BLK1 = R1_SLOT * N1
BLK2 = R2_SLOT * N2
CHUNK = 6400
NCH1 = E1 // CHUNK
E2_PAD = 44800
NCH2 = E2_PAD // CHUNK
DST_PAD_VAL = 800000
SE1_GCAP = 157
SE2_GCAP = 79
SE1_ROWS = 2560
SE1_W = SE1_ROWS * DE
SE2_W = N2 * DE
EA_OFF = 41024

_MESH = plsc.VectorSubcoreMesh(core_axis_name="c", subcore_axis_name="s")


@functools.partial(
    pl.kernel,
    out_type=(
        jax.ShapeDtypeStruct((A1_ROWS * N1,), jnp.float32),
        jax.ShapeDtypeStruct((A2_ROWS * N2,), jnp.float32),
        jax.ShapeDtypeStruct((NW * SE1_W,), jnp.float32),
        jax.ShapeDtypeStruct((NW * SE2_W,), jnp.float32),
    ),
    mesh=_MESH,
    scratch_types=[
        pltpu.VMEM((102400,), jnp.float32),
        pltpu.VMEM((2 * CHUNK,), jnp.int32),
        pltpu.VMEM((2 * CHUNK,), jnp.int32),
        pltpu.SemaphoreType.DMA,
        pltpu.SemaphoreType.DMA,
    ],
    compiler_params=pltpu.CompilerParams(needs_layout_passes=False),
)
def _sc_build(src1, dst1, ea1, src2, dst2, ea2, a1_out, a2_out, se1_out, se2_out,
              buf, idx_s, idx_d, sem_a, sem_b):
    wid = lax.axis_index("s") * NC + lax.axis_index("c")
    zeros16 = jnp.zeros((16,), jnp.float32)
    ones16 = jnp.ones((16,), jnp.float32)
    lane = lax.iota(jnp.int32, 16)

    def zero_buf(nwords):
        def body(i, _):
            for u in range(10):
                buf[pl.ds(i * 160 + u * 16, 16)] = zeros16
            return 0
        lax.fori_loop(0, nwords // 160, body, 0)

    def adj_phase(src, dst, nchunks, ncols, blk, npasses, out_ref):
        ublk = jnp.uint32(blk)

        def scan(base, lo):
            def vec_body(k, _):
                kb = base + k * 64
                for u in range(4):
                    s = idx_s[pl.ds(kb + u * 16, 16)]
                    d = idx_d[pl.ds(kb + u * 16, 16)]
                    flat = d * ncols + s - lo
                    valid = plsc.bitcast(flat, jnp.uint32) < ublk
                    plsc.addupdate_scatter(buf.at[pl.ds(0, blk)], [flat],
                                           ones16, mask=valid)
                return 0
            lax.fori_loop(0, CHUNK // 64, vec_body, 0)

        def issue(c, half):
            sem = sem_a if half == 0 else sem_b
            pltpu.async_copy(src.at[pl.ds(c * CHUNK, CHUNK)],
                             idx_s.at[pl.ds(half * CHUNK, CHUNK)], sem)
            pltpu.async_copy(dst.at[pl.ds(c * CHUNK, CHUNK)],
                             idx_d.at[pl.ds(half * CHUNK, CHUNK)], sem)

        def drain(c, half):
            sem = sem_a if half == 0 else sem_b
            pltpu.make_async_copy(src.at[pl.ds(c * CHUNK, CHUNK)],
                                  idx_s.at[pl.ds(half * CHUNK, CHUNK)],
                                  sem).wait()
            pltpu.make_async_copy(dst.at[pl.ds(c * CHUNK, CHUNK)],
                                  idx_d.at[pl.ds(half * CHUNK, CHUNK)],
                                  sem).wait()

        def pass_body(p, _):
            slot = wid + p * NW
            lo = slot * blk
            issue(0, 0)
            zero_buf(blk)

            def chunk_body(c, _):
                nxt = c + 1

                @pl.when(nxt < nchunks)
                def _():
                    @pl.when(c % 2 == 0)
                    def _():
                        issue(nxt, 1)

                    @pl.when(c % 2 == 1)
                    def _():
                        issue(nxt, 0)

                @pl.when(c % 2 == 0)
                def _():
                    drain(c, 0)
                    scan(0, lo)

                @pl.when(c % 2 == 1)
                def _():
                    drain(c, 1)
                    scan(CHUNK, lo)

                return 0
            lax.fori_loop(0, nchunks, chunk_body, 0)
            pltpu.sync_copy(buf.at[pl.ds(0, blk)],
                            out_ref.at[pl.ds(slot * blk, blk)])
            return 0
        lax.fori_loop(0, npasses, pass_body, 0)

    adj_phase(src1, dst1, NCH1, N1, BLK1, 2, a1_out)
    adj_phase(src2, dst2, NCH2, N2, BLK2, 1, a2_out)

    def se_phase(dst, ea, groups, gcap, nchunk, priv_w, out_ref):
        g0 = (wid * groups) // NW
        g1 = ((wid + 1) * groups) // NW
        ng = g1 - g0
        zero_buf(priv_w)
        priv = buf.at[pl.ds(0, priv_w)]
        ea_stage = buf.at[pl.ds(EA_OFF, gcap * 256)]
        for c in range(nchunk):
            gbase = g0 + c * gcap
            pltpu.sync_copy(dst.at[pl.ds(gbase * 16, gcap * 16)],
                            idx_d.at[pl.ds(0, gcap * 16)])
            pltpu.sync_copy(ea.at[pl.ds(gbase * 256, gcap * 256)], ea_stage)

            def g_body(k, _):
                d = idx_d[pl.ds(k * 16, 16)]
                rowbase = d * DE
                valid = jnp.broadcast_to(c * gcap + k < ng, (16,))
                ebase = k * 256 + lane * 16
                for cc in range(DE):
                    vals = plsc.load_gather(ea_stage, [ebase + cc])
                    plsc.addupdate_scatter(priv, [rowbase + cc], vals,
                                           mask=valid)
                return 0
            lax.fori_loop(0, gcap, g_body, 0)
        pltpu.sync_copy(priv, out_ref.at[pl.ds(wid * priv_w, priv_w)])

    se_phase(dst1, ea1, G1, SE1_GCAP, 2, SE1_W, se1_out)
    se_phase(dst2, ea2, G2, SE2_GCAP, 1, SE2_W, se2_out)


_TC1_BLK = 512
_TC1_GRID = A1_ROWS // _TC1_BLK


def _bdot(a, b):
    return jnp.dot(a.astype(jnp.bfloat16), b.astype(jnp.bfloat16),
                   preferred_element_type=jnp.float32)


def _tc1_body(a1, x_full, x_blk, se1p, wn1x, wn1e, wself, b1, gsc, beta, h_out):
    A = a1[...]
    S = _bdot(A, x_full[...])
    cnt = jnp.sum(A, axis=1, keepdims=True)
    inv = 1.0 / jnp.maximum(cnt, 1.0)
    se = jnp.sum(se1p[...], axis=0)
    t = _bdot(S * inv, wn1x[...])
    t = t + jnp.dot(se * inv, wn1e[...], preferred_element_type=jnp.float32)
    t = t + _bdot(x_blk[...], wself[...])
    t = t + b1[...]
    h_out[...] = jnp.maximum(t, 0.0) * gsc[...] + beta[...]


_tc1 = pl.pallas_call(
    _tc1_body,
    grid=(_TC1_GRID,),
    in_specs=[
        pl.BlockSpec((_TC1_BLK, N1), lambda i: (i, 0)),
        pl.BlockSpec((N1, 256), lambda i: (0, 0)),
        pl.BlockSpec((_TC1_BLK, 256), lambda i: (i, 0)),
        pl.BlockSpec((NW, _TC1_BLK, DE), lambda i: (0, i, 0)),
        pl.BlockSpec((256, 1500), lambda i: (0, 0)),
        pl.BlockSpec((DE, 1500), lambda i: (0, 0)),
        pl.BlockSpec((256, 1500), lambda i: (0, 0)),
        pl.BlockSpec((1, 1500), lambda i: (0, 0)),
        pl.BlockSpec((1, 1500), lambda i: (0, 0)),
        pl.BlockSpec((1, 1500), lambda i: (0, 0)),
    ],
    out_specs=pl.BlockSpec((_TC1_BLK, 1500), lambda i: (i, 0)),
    out_shape=jax.ShapeDtypeStruct((A1_ROWS, 1500), jnp.float32),
    compiler_params=pltpu.CompilerParams(dimension_semantics=("parallel",)),
)


def _tc2_body(a2, h1k, se2p, wn2h, wn2e, wself2, b2, out):
    A = a2[...]
    H = h1k[...]
    P = _bdot(H, wn2h[...])
    G = _bdot(A, P)
    cnt = jnp.sum(A, axis=1, keepdims=True)
    inv = 1.0 / jnp.maximum(cnt, 1.0)
    se = jnp.sum(se2p[...], axis=0)
    o = (G * inv
         + jnp.dot(se * inv, wn2e[...], preferred_element_type=jnp.float32)
         + _bdot(H, wself2[...])
         + b2[...])
    m = jnp.max(o, axis=1, keepdims=True)
    e = jnp.exp(o - m)
    out[...] = (o - m) - jnp.log(jnp.sum(e, axis=1, keepdims=True))


_tc2 = pl.pallas_call(
    _tc2_body,
    out_shape=jax.ShapeDtypeStruct((N2, 128), jnp.float32),
)


def kernel(x, res_size1, edge_index1, edge_attr1, res_size2, edge_index2,
           edge_attr2, W_nbr1, W_self1, b1, gamma, beta, W_nbr2, W_self2, b2):
    src1 = jnp.pad(edge_index1[0].astype(jnp.int32), (0, 6400))
    dst1 = jnp.pad(edge_index1[1].astype(jnp.int32), (0, 6400),
                   constant_values=DST_PAD_VAL)
    src2 = jnp.pad(edge_index2[0].astype(jnp.int32), (0, E2_PAD - E2))
    dst2 = jnp.pad(edge_index2[1].astype(jnp.int32), (0, E2_PAD - E2),
                   constant_values=DST_PAD_VAL)
    ea1 = jnp.pad(edge_attr1.reshape(-1), (0, 512 * DE))
    ea2 = edge_attr2.reshape(-1)

    a1f, a2f, se1f, se2f = _sc_build(src1, dst1, ea1, src2, dst2, ea2)
    A1 = a1f.reshape(A1_ROWS, N1)
    A2 = a2f.reshape(A2_ROWS, N2)[:N2]
    SE1 = se1f.reshape(NW, SE1_ROWS, DE)
    SE2 = se2f.reshape(NW, N2, DE)

    x25 = jnp.pad(x[:N1], ((0, A1_ROWS - N1), (0, 0)))
    gscale = (gamma * (1.0 / jnp.sqrt(jnp.float32(1.0 + 1e-5))))[None]

    h = _tc1(A1, x25[:N1], x25, SE1, W_nbr1[:256], W_nbr1[256:], W_self1,
             b1[None], gscale, beta[None])
    out = _tc2(A2, h[:N2], SE2, W_nbr2[:1500], W_nbr2[1500:], W_self2,
               b2[None])
    return out


# speedup vs baseline: 5.5050x; 1.3053x over previous
import functools

import jax
import jax.numpy as jnp
from jax import lax
from jax.experimental import pallas as pl
from jax.experimental.pallas import tpu as pltpu
from jax.experimental.pallas import tpu_sc as plsc

NC, NS, LANES = 2, 16, 16
NW = NC * NS

N1, N2 = 2500, 1000
E1, E2 = 160000, 40000
G1, G2 = E1 // 16, E2 // 16
DE = 16

A1_ROWS, A2_ROWS = 2560, 1024
A1_SLOT_ROWS = 256
A2_SLOT_ROWS = 512
REG1 = A1_SLOT_ROWS * N1
REG2 = A2_SLOT_ROWS * N2
ASH = REG1 + 2560
ZSTRIPE = ASH // 80
ECH = 2000
EPT1 = E1 // NS
E2_PAD = 64000
EPT2 = E2_PAD // NS
DST_PAD_VAL = 800000
SE1_GCAP = 157
SE2_GCAP = 79
SE1_ROWS = 2560
SE1_W = SE1_ROWS * DE
SE2_W = N2 * DE
EA_OFF = 41024

_MESH = plsc.VectorSubcoreMesh(core_axis_name="c", subcore_axis_name="s")


@functools.partial(
    pl.kernel,
    out_type=(
        jax.ShapeDtypeStruct((A1_ROWS * N1,), jnp.float32),
        jax.ShapeDtypeStruct((A2_ROWS * N2,), jnp.float32),
        jax.ShapeDtypeStruct((NW * SE1_W,), jnp.float32),
        jax.ShapeDtypeStruct((NW * SE2_W,), jnp.float32),
    ),
    mesh=_MESH,
    scratch_types=[
        pltpu.VMEM((81280,), jnp.float32),
        pltpu.VMEM((ECH,), jnp.int32),
        pltpu.VMEM((ECH,), jnp.int32),
        pltpu.VMEM((ECH,), jnp.int32),
        pltpu.VMEM((ECH,), jnp.float32),
        pltpu.VMEM_SHARED((ASH,), jnp.float32),
    ],
    compiler_params=pltpu.CompilerParams(needs_layout_passes=False),
)
def _sc_build(src1, dst1, ea1, src2, dst2, ea2, a1_out, a2_out, se1_out, se2_out,
              buf, idx_s, idx_d, idx_w, ones_v, ash):
    core = lax.axis_index("c")
    tid = lax.axis_index("s")
    wid = tid * NC + core
    zeros16 = jnp.zeros((16,), jnp.float32)
    ones16 = jnp.ones((16,), jnp.float32)
    lane = lax.iota(jnp.int32, 16)

    def zero_buf(nwords):
        def body(i, _):
            for u in range(10):
                buf[pl.ds(i * 160 + u * 16, 16)] = zeros16
            return 0
        lax.fori_loop(0, nwords // 160, body, 0)

    zero_buf(8160)

    def fill_ones(i, _):
        ones_v[pl.ds(i * 16, 16)] = ones16
        return 0
    lax.fori_loop(0, ECH // 16, fill_ones, 0)

    def adj_round(src, dst, ept, ncols, region, slot, iw, out_ref):
        lo = slot * region
        ureg = jnp.uint32(region)
        for z in range(5):
            pltpu.sync_copy(buf.at[pl.ds(0, ZSTRIPE)],
                            ash.at[pl.ds((tid * 5 + z) * ZSTRIPE, ZSTRIPE)])
        plsc.subcore_barrier()

        def chunk_body(c, _):
            tbase = tid * ept + c * ECH
            pltpu.sync_copy(src.at[pl.ds(tbase, ECH)], idx_s)
            pltpu.sync_copy(dst.at[pl.ds(tbase, ECH)], idx_d)

            def vec_body(k, _):
                s = idx_s[pl.ds(k * 16, 16)]
                d = idx_d[pl.ds(k * 16, 16)]
                local = d * ncols + s - lo
                valid = plsc.bitcast(local, jnp.uint32) < ureg
                iw[pl.ds(k * 16, 16)] = jnp.where(valid, local, region + s)
                return 0
            lax.fori_loop(0, ECH // 16, vec_body, 0)
            pltpu.sync_copy(ones_v, ash.at[iw], add=True)
            return 0
        lax.fori_loop(0, ept // ECH, chunk_body, 0)
        plsc.subcore_barrier()
        ostripe = region // 16
        sub = region // 80
        for z in range(5):
            off = tid * ostripe + z * sub
            pltpu.sync_copy(ash.at[pl.ds(off, sub)],
                            buf.at[pl.ds(10040, sub)])
            pltpu.sync_copy(buf.at[pl.ds(10040, sub)],
                            out_ref.at[pl.ds(lo + off, sub)])
        plsc.subcore_barrier()

    def a1_round(r, _):
        adj_round(src1, dst1, EPT1, N1, REG1, r * NC + core, idx_w, a1_out)
        return 0
    lax.fori_loop(0, 5, a1_round, 0)
    adj_round(src2, dst2, EPT2, N2, REG2, core, idx_w, a2_out)

    def se_phase(dst, ea, groups, gcap, nchunk, nrows, out_ref):
        g0 = (wid * groups) // NW
        g1 = ((wid + 1) * groups) // NW
        priv_w = nrows * DE
        zero_buf(priv_w)
        priv = buf.at[pl.ds(0, priv_w)]
        ea_stage = buf.at[pl.ds(EA_OFF, gcap * 256)]
        for c in range(nchunk):
            start = g0 + c * gcap
            gb = jnp.minimum(start, groups - gcap)
            pltpu.sync_copy(dst.at[pl.ds(gb * 16, gcap * 16)],
                            idx_d.at[pl.ds(0, gcap * 16)])
            pltpu.sync_copy(ea.at[pl.ds(gb * 256, gcap * 256)], ea_stage)

            def g_body(k, _):
                d = idx_d[pl.ds(k * 16, 16)]
                rowbase = d * DE
                g = gb + k
                valid = jnp.broadcast_to((g >= start) & (g < g1), (16,))
                ebase = k * 256 + lane * 16
                for cc in range(DE):
                    vals = plsc.load_gather(ea_stage, [ebase + cc])
                    plsc.addupdate_scatter(priv, [rowbase + cc], vals,
                                           mask=valid)
                return 0
            lax.fori_loop(0, gcap, g_body, 0)
        pltpu.sync_copy(priv, out_ref.at[pl.ds(wid * priv_w, priv_w)])

    se_phase(dst1, ea1, G1, SE1_GCAP, 2, SE1_ROWS, se1_out)
    se_phase(dst2, ea2, G2, SE2_GCAP, 1, N2, se2_out)


_TC1_BLK = 512
_TC1_GRID = A1_ROWS // _TC1_BLK


def _bdot(a, b):
    return jnp.dot(a.astype(jnp.bfloat16), b.astype(jnp.bfloat16),
                   preferred_element_type=jnp.float32)


def _tc1_body(a1, x_full, x_blk, se1p, wn1x, wn1e, wself, b1, gsc, beta, h_out):
    A = a1[...]
    S = _bdot(A, x_full[...])
    cnt = jnp.sum(A, axis=1, keepdims=True)
    inv = 1.0 / jnp.maximum(cnt, 1.0)
    se = jnp.sum(se1p[...], axis=0)
    t = _bdot(S * inv, wn1x[...])
    t = t + jnp.dot(se * inv, wn1e[...], preferred_element_type=jnp.float32)
    t = t + _bdot(x_blk[...], wself[...])
    t = t + b1[...]
    h_out[...] = jnp.maximum(t, 0.0) * gsc[...] + beta[...]


_tc1 = pl.pallas_call(
    _tc1_body,
    grid=(_TC1_GRID,),
    in_specs=[
        pl.BlockSpec((_TC1_BLK, N1), lambda i: (i, 0)),
        pl.BlockSpec((N1, 256), lambda i: (0, 0)),
        pl.BlockSpec((_TC1_BLK, 256), lambda i: (i, 0)),
        pl.BlockSpec((NW, _TC1_BLK, DE), lambda i: (0, i, 0)),
        pl.BlockSpec((256, 1500), lambda i: (0, 0)),
        pl.BlockSpec((DE, 1500), lambda i: (0, 0)),
        pl.BlockSpec((256, 1500), lambda i: (0, 0)),
        pl.BlockSpec((1, 1500), lambda i: (0, 0)),
        pl.BlockSpec((1, 1500), lambda i: (0, 0)),
        pl.BlockSpec((1, 1500), lambda i: (0, 0)),
    ],
    out_specs=pl.BlockSpec((_TC1_BLK, 1500), lambda i: (i, 0)),
    out_shape=jax.ShapeDtypeStruct((A1_ROWS, 1500), jnp.float32),
    compiler_params=pltpu.CompilerParams(dimension_semantics=("parallel",)),
)


def _tc2_body(a2, h1k, se2p, wn2h, wn2e, wself2, b2, out):
    A = a2[...]
    H = h1k[...]
    P = _bdot(H, wn2h[...])
    G = _bdot(A, P)
    cnt = jnp.sum(A, axis=1, keepdims=True)
    inv = 1.0 / jnp.maximum(cnt, 1.0)
    se = jnp.sum(se2p[...], axis=0)
    o = (G * inv
         + jnp.dot(se * inv, wn2e[...], preferred_element_type=jnp.float32)
         + _bdot(H, wself2[...])
         + b2[...])
    m = jnp.max(o, axis=1, keepdims=True)
    e = jnp.exp(o - m)
    out[...] = (o - m) - jnp.log(jnp.sum(e, axis=1, keepdims=True))


_tc2 = pl.pallas_call(
    _tc2_body,
    out_shape=jax.ShapeDtypeStruct((N2, 128), jnp.float32),
)


def kernel(x, res_size1, edge_index1, edge_attr1, res_size2, edge_index2,
           edge_attr2, W_nbr1, W_self1, b1, gamma, beta, W_nbr2, W_self2, b2):
    src1 = edge_index1[0].astype(jnp.int32)
    dst1 = jnp.pad(edge_index1[1].astype(jnp.int32), (0, 512))
    src2 = jnp.concatenate([edge_index2[0].astype(jnp.int32),
                            jnp.arange(E2_PAD - E2, dtype=jnp.int32) % N2])
    dst2 = jnp.pad(edge_index2[1].astype(jnp.int32), (0, E2_PAD - E2),
                   constant_values=DST_PAD_VAL)
    ea1 = jnp.pad(edge_attr1.reshape(-1), (0, 512 * DE))
    ea2 = edge_attr2.reshape(-1)

    a1f, a2f, se1f, se2f = _sc_build(src1, dst1, ea1, src2, dst2, ea2)
    A1 = a1f.reshape(A1_ROWS, N1)
    A2 = a2f.reshape(A2_ROWS, N2)[:N2]
    SE1 = se1f.reshape(NW, SE1_ROWS, DE)
    SE2 = se2f.reshape(NW, N2, DE)

    x25 = jnp.pad(x[:N1], ((0, A1_ROWS - N1), (0, 0)))
    gscale = (gamma * (1.0 / jnp.sqrt(jnp.float32(1.0 + 1e-5))))[None]

    h = _tc1(A1, x25[:N1], x25, SE1, W_nbr1[:256], W_nbr1[256:], W_self1,
             b1[None], gscale, beta[None])
    out = _tc2(A2, h[:N2], SE2, W_nbr2[:1500], W_nbr2[1500:], W_self2,
               b2[None])
    return out


# speedup vs baseline: 5.5980x; 1.0169x over previous
import functools

import jax
import jax.numpy as jnp
from jax import lax
from jax.experimental import pallas as pl
from jax.experimental.pallas import tpu as pltpu
from jax.experimental.pallas import tpu_sc as plsc

NC, NS, LANES = 2, 16, 16
NW = NC * NS

N1, N2 = 2500, 1000
E1, E2 = 160000, 40000
G1, G2 = E1 // 16, E2 // 16
DE = 16

A1_ROWS, A2_ROWS = 2560, 1024
A1_SLOT_ROWS = 256
A2_SLOT_ROWS = 512
REG1 = A1_SLOT_ROWS * N1
REG2 = A2_SLOT_ROWS * N2
ASH = REG1 + 2560
ZSTRIPE = ASH // 80
ECH = 2000
EPT1 = E1 // NS
E2_PAD = 64000
EPT2 = E2_PAD // NS
DST_PAD_VAL = 800000
SE1_GCAP = 157
SE2_GCAP = 79
SE1_ROWS = 2560
SE1_W = SE1_ROWS * DE
SE2_W = N2 * DE
EA_OFF = 41024

_MESH = plsc.VectorSubcoreMesh(core_axis_name="c", subcore_axis_name="s")


@functools.partial(
    pl.kernel,
    out_type=(
        jax.ShapeDtypeStruct((A1_ROWS * N1,), jnp.float32),
        jax.ShapeDtypeStruct((A2_ROWS * N2,), jnp.float32),
        jax.ShapeDtypeStruct((NW * SE1_W,), jnp.float32),
        jax.ShapeDtypeStruct((NW * SE2_W,), jnp.float32),
    ),
    mesh=_MESH,
    scratch_types=[
        pltpu.VMEM((81280,), jnp.float32),
        pltpu.VMEM((ECH,), jnp.int32),
        pltpu.VMEM((ECH,), jnp.int32),
        pltpu.VMEM((ECH,), jnp.int32),
        pltpu.VMEM((ECH,), jnp.float32),
        pltpu.VMEM_SHARED((ASH,), jnp.float32),
    ],
    compiler_params=pltpu.CompilerParams(needs_layout_passes=False),
)
def _sc_build(src1, dst1, ea1, src2, dst2, ea2, a1_out, a2_out, se1_out, se2_out,
              buf, idx_s, idx_d, idx_w, ones_v, ash):
    core = lax.axis_index("c")
    tid = lax.axis_index("s")
    wid = tid * NC + core
    zeros16 = jnp.zeros((16,), jnp.float32)
    ones16 = jnp.ones((16,), jnp.float32)
    lane = lax.iota(jnp.int32, 16)

    def zero_buf(nwords):
        def body(i, _):
            for u in range(10):
                buf[pl.ds(i * 160 + u * 16, 16)] = zeros16
            return 0
        lax.fori_loop(0, nwords // 160, body, 0)

    zero_buf(8160)

    def fill_ones(i, _):
        ones_v[pl.ds(i * 16, 16)] = ones16
        return 0
    lax.fori_loop(0, ECH // 16, fill_ones, 0)

    def adj_round(src, dst, ept, ncols, region, slot, iw, out_ref):
        lo = slot * region
        ureg = jnp.uint32(region)
        for z in range(5):
            pltpu.sync_copy(buf.at[pl.ds(0, ZSTRIPE)],
                            ash.at[pl.ds((tid * 5 + z) * ZSTRIPE, ZSTRIPE)])
        plsc.subcore_barrier()

        def chunk_body(c, _):
            tbase = tid * ept + c * ECH
            pltpu.sync_copy(src.at[pl.ds(tbase, ECH)], idx_s)
            pltpu.sync_copy(dst.at[pl.ds(tbase, ECH)], idx_d)

            def vec_body(k, _):
                s = idx_s[pl.ds(k * 16, 16)]
                d = idx_d[pl.ds(k * 16, 16)]
                local = d * ncols + s - lo
                valid = plsc.bitcast(local, jnp.uint32) < ureg
                iw[pl.ds(k * 16, 16)] = jnp.where(valid, local, region + s)
                return 0
            lax.fori_loop(0, ECH // 16, vec_body, 0)
            pltpu.sync_copy(ones_v, ash.at[iw], add=True)
            return 0
        lax.fori_loop(0, ept // ECH, chunk_body, 0)
        plsc.subcore_barrier()
        ostripe = region // 16
        sub = region // 80
        for z in range(5):
            off = tid * ostripe + z * sub
            pltpu.sync_copy(ash.at[pl.ds(off, sub)],
                            buf.at[pl.ds(10040, sub)])
            pltpu.sync_copy(buf.at[pl.ds(10040, sub)],
                            out_ref.at[pl.ds(lo + off, sub)])
        plsc.subcore_barrier()

    def a1_round(r, _):
        adj_round(src1, dst1, EPT1, N1, REG1, r * NC + core, idx_w, a1_out)
        return 0
    lax.fori_loop(0, 5, a1_round, 0)
    adj_round(src2, dst2, EPT2, N2, REG2, core, idx_w, a2_out)

    def se_phase(dst, ea, groups, gcap, nchunk, nrows, out_ref):
        g0 = (wid * groups) // NW
        g1 = ((wid + 1) * groups) // NW
        priv_w = nrows * DE
        zero_buf(priv_w)
        priv = buf.at[pl.ds(0, priv_w)]
        ea_stage = buf.at[pl.ds(EA_OFF, gcap * 256)]
        for c in range(nchunk):
            start = g0 + c * gcap
            gb = jnp.minimum(start, groups - gcap)
            pltpu.sync_copy(dst.at[pl.ds(gb * 16, gcap * 16)],
                            idx_d.at[pl.ds(0, gcap * 16)])
            pltpu.sync_copy(ea.at[pl.ds(gb * 256, gcap * 256)], ea_stage)

            def g_body(k, _):
                d = idx_d[pl.ds(k * 16, 16)]
                rowbase = d * DE
                g = gb + k
                valid = jnp.broadcast_to((g >= start) & (g < g1), (16,))
                ebase = k * 256 + lane * 16
                for cc in range(DE):
                    vals = plsc.load_gather(ea_stage, [ebase + cc])
                    plsc.addupdate_scatter(priv, [rowbase + cc], vals,
                                           mask=valid)
                return 0
            lax.fori_loop(0, gcap, g_body, 0)
        pltpu.sync_copy(priv, out_ref.at[pl.ds(wid * priv_w, priv_w)])

    se_phase(dst1, ea1, G1, SE1_GCAP, 2, SE1_ROWS, se1_out)
    se_phase(dst2, ea2, G2, SE2_GCAP, 1, N2, se2_out)


_TC1_BLK = 512
_TC1_GRID = A1_ROWS // _TC1_BLK


def _bdot(a, b):
    return jnp.dot(a.astype(jnp.bfloat16), b.astype(jnp.bfloat16),
                   preferred_element_type=jnp.float32)


def _tc1_body(a1, x_full, x_blk, se1p, wn1x, wn1e, wself, b1, gsc, beta, h_out):
    A = a1[...]
    S = _bdot(A, x_full[...])
    cnt = jnp.sum(A, axis=1, keepdims=True)
    inv = 1.0 / jnp.maximum(cnt, 1.0)
    se = jnp.sum(se1p[...], axis=0)
    t = _bdot(S * inv, wn1x[...])
    t = t + jnp.dot(se * inv, wn1e[...], preferred_element_type=jnp.float32)
    t = t + _bdot(x_blk[...], wself[...])
    t = t + b1[...]
    h_out[...] = jnp.maximum(t, 0.0) * gsc[...] + beta[...]


_tc1 = pl.pallas_call(
    _tc1_body,
    grid=(_TC1_GRID,),
    in_specs=[
        pl.BlockSpec((_TC1_BLK, N1), lambda i: (i, 0)),
        pl.BlockSpec((N1, 256), lambda i: (0, 0)),
        pl.BlockSpec((_TC1_BLK, 256), lambda i: (i, 0)),
        pl.BlockSpec((NW, _TC1_BLK, DE), lambda i: (0, i, 0)),
        pl.BlockSpec((256, 1500), lambda i: (0, 0)),
        pl.BlockSpec((DE, 1500), lambda i: (0, 0)),
        pl.BlockSpec((256, 1500), lambda i: (0, 0)),
        pl.BlockSpec((1, 1500), lambda i: (0, 0)),
        pl.BlockSpec((1, 1500), lambda i: (0, 0)),
        pl.BlockSpec((1, 1500), lambda i: (0, 0)),
    ],
    out_specs=pl.BlockSpec((_TC1_BLK, 1500), lambda i: (i, 0)),
    out_shape=jax.ShapeDtypeStruct((A1_ROWS, 1500), jnp.float32),
    compiler_params=pltpu.CompilerParams(dimension_semantics=("parallel",)),
)


def _tc2_body(a2, h1k, se2p, wn2h, wn2e, wself2, b2, out):
    A = a2[...]
    H = h1k[...]
    P = _bdot(H, wn2h[...])
    G = _bdot(A, P)
    cnt = jnp.sum(A, axis=1, keepdims=True)
    inv = 1.0 / jnp.maximum(cnt, 1.0)
    se = jnp.sum(se2p[...], axis=0)
    o = (G * inv
         + jnp.dot(se * inv, wn2e[...], preferred_element_type=jnp.float32)
         + _bdot(H, wself2[...])
         + b2[...])
    m = jnp.max(o, axis=1, keepdims=True)
    e = jnp.exp(o - m)
    out[...] = (o - m) - jnp.log(jnp.sum(e, axis=1, keepdims=True))


_tc2 = pl.pallas_call(
    _tc2_body,
    out_shape=jax.ShapeDtypeStruct((N2, 128), jnp.float32),
)


def kernel(x, res_size1, edge_index1, edge_attr1, res_size2, edge_index2,
           edge_attr2, W_nbr1, W_self1, b1, gamma, beta, W_nbr2, W_self2, b2):
    src1 = edge_index1[0].astype(jnp.int32)
    dst1 = edge_index1[1].astype(jnp.int32)
    src2 = jnp.concatenate([edge_index2[0].astype(jnp.int32),
                            jnp.arange(E2_PAD - E2, dtype=jnp.int32) % N2])
    dst2 = jnp.pad(edge_index2[1].astype(jnp.int32), (0, E2_PAD - E2),
                   constant_values=DST_PAD_VAL)
    ea1 = edge_attr1.reshape(-1)
    ea2 = edge_attr2.reshape(-1)

    a1f, a2f, se1f, se2f = _sc_build(src1, dst1, ea1, src2, dst2, ea2)
    A1 = a1f.reshape(A1_ROWS, N1)
    A2 = a2f.reshape(A2_ROWS, N2)[:N2]
    SE1 = se1f.reshape(NW, SE1_ROWS, DE)
    SE2 = se2f.reshape(NW, N2, DE)

    x25 = jnp.pad(x[:N1], ((0, A1_ROWS - N1), (0, 0)))
    gscale = (gamma * (1.0 / jnp.sqrt(jnp.float32(1.0 + 1e-5))))[None]

    h = _tc1(A1, x25[:N1], x25, SE1, W_nbr1[:256], W_nbr1[256:], W_self1,
             b1[None], gscale, beta[None])
    out = _tc2(A2, h[:N2], SE2, W_nbr2[:1500], W_nbr2[1500:], W_self2,
               b2[None])
    return out


# speedup vs baseline: 6.9604x; 1.2434x over previous
import functools

import jax
import jax.numpy as jnp
from jax import lax
from jax.experimental import pallas as pl
from jax.experimental.pallas import tpu as pltpu
from jax.experimental.pallas import tpu_sc as plsc

NC, NS, LANES = 2, 16, 16
NW = NC * NS

N1, N2 = 2500, 1000
E1, E2 = 160000, 40000
G1, G2 = E1 // 16, E2 // 16
DE = 16

A1_ROWS, A2_ROWS = 2560, 1024
A1_SLOT_ROWS = 256
A2_SLOT_ROWS = 512
REG1 = A1_SLOT_ROWS * N1
REG2 = A2_SLOT_ROWS * N2
ASH = REG1 + 2560
ZSTRIPE = ASH // 80
ECH = 2000
EPT1 = E1 // NS
E2_PAD = 64000
EPT2 = E2_PAD // NS
DST_PAD_VAL = 800000
SE1_GCAP = 157
SE2_GCAP = 79
SE1_ROWS = 2560
SE1_W = SE1_ROWS * DE
SE2_W = N2 * DE
EA_OFF = 41024

_MESH = plsc.VectorSubcoreMesh(core_axis_name="c", subcore_axis_name="s")


@functools.partial(
    pl.kernel,
    out_type=(
        jax.ShapeDtypeStruct((A1_ROWS * N1,), jnp.float32),
        jax.ShapeDtypeStruct((A2_ROWS * N2,), jnp.float32),
    ),
    mesh=_MESH,
    scratch_types=[
        pltpu.VMEM((18080,), jnp.float32),
        pltpu.VMEM((ECH,), jnp.int32),
        pltpu.VMEM((ECH,), jnp.int32),
        pltpu.VMEM((ECH,), jnp.int32),
        pltpu.VMEM((ECH,), jnp.float32),
        pltpu.VMEM_SHARED((ASH,), jnp.float32),
    ],
    compiler_params=pltpu.CompilerParams(needs_layout_passes=False),
)
def _sc_adj(src1, dst1, src2, dst2, a1_out, a2_out,
            buf, idx_s, idx_d, idx_w, ones_v, ash):
    core = lax.axis_index("c")
    tid = lax.axis_index("s")
    zeros16 = jnp.zeros((16,), jnp.float32)
    ones16 = jnp.ones((16,), jnp.float32)

    def zero_buf(nwords):
        def body(i, _):
            for u in range(10):
                buf[pl.ds(i * 160 + u * 16, 16)] = zeros16
            return 0
        lax.fori_loop(0, nwords // 160, body, 0)

    zero_buf(8160)

    def fill_ones(i, _):
        ones_v[pl.ds(i * 16, 16)] = ones16
        return 0
    lax.fori_loop(0, ECH // 16, fill_ones, 0)

    def adj_round(src, dst, ept, ncols, region, slot, iw, out_ref):
        lo = slot * region
        ureg = jnp.uint32(region)
        for z in range(5):
            pltpu.sync_copy(buf.at[pl.ds(0, ZSTRIPE)],
                            ash.at[pl.ds((tid * 5 + z) * ZSTRIPE, ZSTRIPE)])
        plsc.subcore_barrier()

        def chunk_body(c, _):
            tbase = tid * ept + c * ECH
            pltpu.sync_copy(src.at[pl.ds(tbase, ECH)], idx_s)
            pltpu.sync_copy(dst.at[pl.ds(tbase, ECH)], idx_d)

            def vec_body(k, _):
                s = idx_s[pl.ds(k * 16, 16)]
                d = idx_d[pl.ds(k * 16, 16)]
                local = d * ncols + s - lo
                valid = plsc.bitcast(local, jnp.uint32) < ureg
                iw[pl.ds(k * 16, 16)] = jnp.where(valid, local, region + s)
                return 0
            lax.fori_loop(0, ECH // 16, vec_body, 0)
            pltpu.sync_copy(ones_v, ash.at[iw], add=True)
            return 0
        lax.fori_loop(0, ept // ECH, chunk_body, 0)
        plsc.subcore_barrier()
        ostripe = region // 16
        sub = region // 80
        for z in range(5):
            off = tid * ostripe + z * sub
            pltpu.sync_copy(ash.at[pl.ds(off, sub)],
                            buf.at[pl.ds(10040, sub)])
            pltpu.sync_copy(buf.at[pl.ds(10040, sub)],
                            out_ref.at[pl.ds(lo + off, sub)])
        plsc.subcore_barrier()

    def a1_round(r, _):
        adj_round(src1, dst1, EPT1, N1, REG1, r * NC + core, idx_w, a1_out)
        return 0
    lax.fori_loop(0, 5, a1_round, 0)
    adj_round(src2, dst2, EPT2, N2, REG2, core, idx_w, a2_out)


@functools.partial(
    pl.kernel,
    out_type=(
        jax.ShapeDtypeStruct((NW * SE1_W,), jnp.float32),
        jax.ShapeDtypeStruct((NW * SE2_W,), jnp.float32),
    ),
    mesh=_MESH,
    scratch_types=[
        pltpu.VMEM((81280,), jnp.float32),
        pltpu.VMEM((2512,), jnp.int32),
    ],
    compiler_params=pltpu.CompilerParams(needs_layout_passes=False),
)
def _sc_se(dst1, ea1, dst2, ea2, se1_out, se2_out, buf, idx_d):
    core = lax.axis_index("c")
    tid = lax.axis_index("s")
    wid = tid * NC + core
    zeros16 = jnp.zeros((16,), jnp.float32)
    lane = lax.iota(jnp.int32, 16)

    def zero_buf(nwords):
        def body(i, _):
            for u in range(10):
                buf[pl.ds(i * 160 + u * 16, 16)] = zeros16
            return 0
        lax.fori_loop(0, nwords // 160, body, 0)

    def se_phase(dst, ea, groups, gcap, nchunk, nrows, out_ref):
        g0 = (wid * groups) // NW
        g1 = ((wid + 1) * groups) // NW
        priv_w = nrows * DE
        zero_buf(priv_w)
        priv = buf.at[pl.ds(0, priv_w)]
        ea_stage = buf.at[pl.ds(EA_OFF, gcap * 256)]
        for c in range(nchunk):
            start = g0 + c * gcap
            gb = jnp.minimum(start, groups - gcap)
            pltpu.sync_copy(dst.at[pl.ds(gb * 16, gcap * 16)],
                            idx_d.at[pl.ds(0, gcap * 16)])
            pltpu.sync_copy(ea.at[pl.ds(gb * 256, gcap * 256)], ea_stage)

            def g_body(k, _):
                d = idx_d[pl.ds(k * 16, 16)]
                rowbase = d * DE
                g = gb + k
                valid = jnp.broadcast_to((g >= start) & (g < g1), (16,))
                ebase = k * 256 + lane * 16
                for cc in range(DE):
                    vals = plsc.load_gather(ea_stage, [ebase + cc])
                    plsc.addupdate_scatter(priv, [rowbase + cc], vals,
                                           mask=valid)
                return 0
            lax.fori_loop(0, gcap, g_body, 0)
        pltpu.sync_copy(priv, out_ref.at[pl.ds(wid * priv_w, priv_w)])

    se_phase(dst1, ea1, G1, SE1_GCAP, 2, SE1_ROWS, se1_out)
    se_phase(dst2, ea2, G2, SE2_GCAP, 1, N2, se2_out)


_TC1_BLK = 512
_TC1_GRID = A1_ROWS // _TC1_BLK


def _bdot(a, b):
    return jnp.dot(a.astype(jnp.bfloat16), b.astype(jnp.bfloat16),
                   preferred_element_type=jnp.float32)


def _tc1_body(a1, x_full, x_blk, se1p, wn1x, wn1e, wself, b1, gsc, beta, h_out):
    A = a1[...]
    S = _bdot(A, x_full[...])
    cnt = jnp.sum(A, axis=1, keepdims=True)
    inv = 1.0 / jnp.maximum(cnt, 1.0)
    se = jnp.sum(se1p[...], axis=0)
    t = _bdot(S * inv, wn1x[...])
    t = t + jnp.dot(se * inv, wn1e[...], preferred_element_type=jnp.float32)
    t = t + _bdot(x_blk[...], wself[...])
    t = t + b1[...]
    h_out[...] = jnp.maximum(t, 0.0) * gsc[...] + beta[...]


_tc1 = pl.pallas_call(
    _tc1_body,
    grid=(_TC1_GRID,),
    in_specs=[
        pl.BlockSpec((_TC1_BLK, N1), lambda i: (i, 0)),
        pl.BlockSpec((N1, 256), lambda i: (0, 0)),
        pl.BlockSpec((_TC1_BLK, 256), lambda i: (i, 0)),
        pl.BlockSpec((NW, _TC1_BLK, DE), lambda i: (0, i, 0)),
        pl.BlockSpec((256, 1500), lambda i: (0, 0)),
        pl.BlockSpec((DE, 1500), lambda i: (0, 0)),
        pl.BlockSpec((256, 1500), lambda i: (0, 0)),
        pl.BlockSpec((1, 1500), lambda i: (0, 0)),
        pl.BlockSpec((1, 1500), lambda i: (0, 0)),
        pl.BlockSpec((1, 1500), lambda i: (0, 0)),
    ],
    out_specs=pl.BlockSpec((_TC1_BLK, 1500), lambda i: (i, 0)),
    out_shape=jax.ShapeDtypeStruct((A1_ROWS, 1500), jnp.float32),
    compiler_params=pltpu.CompilerParams(dimension_semantics=("parallel",)),
)


def _tc2_body(a2, h1k, se2p, wn2h, wn2e, wself2, b2, out):
    A = a2[...]
    H = h1k[...]
    P = _bdot(H, wn2h[...])
    G = _bdot(A, P)
    cnt = jnp.sum(A, axis=1, keepdims=True)
    inv = 1.0 / jnp.maximum(cnt, 1.0)
    se = jnp.sum(se2p[...], axis=0)
    o = (G * inv
         + jnp.dot(se * inv, wn2e[...], preferred_element_type=jnp.float32)
         + _bdot(H, wself2[...])
         + b2[...])
    m = jnp.max(o, axis=1, keepdims=True)
    e = jnp.exp(o - m)
    out[...] = (o - m) - jnp.log(jnp.sum(e, axis=1, keepdims=True))


_tc2 = pl.pallas_call(
    _tc2_body,
    out_shape=jax.ShapeDtypeStruct((N2, 128), jnp.float32),
)


def kernel(x, res_size1, edge_index1, edge_attr1, res_size2, edge_index2,
           edge_attr2, W_nbr1, W_self1, b1, gamma, beta, W_nbr2, W_self2, b2):
    src1 = edge_index1[0].astype(jnp.int32)
    dst1 = edge_index1[1].astype(jnp.int32)
    src2 = jnp.concatenate([edge_index2[0].astype(jnp.int32),
                            jnp.arange(E2_PAD - E2, dtype=jnp.int32) % N2])
    dst2 = jnp.pad(edge_index2[1].astype(jnp.int32), (0, E2_PAD - E2),
                   constant_values=DST_PAD_VAL)
    ea1 = edge_attr1.reshape(-1)
    ea2 = edge_attr2.reshape(-1)

    a1f, a2f = _sc_adj(src1, dst1, src2, dst2)
    se1f, se2f = _sc_se(dst1, ea1, dst2, ea2)
    A1 = a1f.reshape(A1_ROWS, N1)
    A2 = a2f.reshape(A2_ROWS, N2)[:N2]
    SE1 = se1f.reshape(NW, SE1_ROWS, DE)
    SE2 = se2f.reshape(NW, N2, DE)

    x25 = jnp.pad(x[:N1], ((0, A1_ROWS - N1), (0, 0)))
    gscale = (gamma * (1.0 / jnp.sqrt(jnp.float32(1.0 + 1e-5))))[None]

    h = _tc1(A1, x25[:N1], x25, SE1, W_nbr1[:256], W_nbr1[256:], W_self1,
             b1[None], gscale, beta[None])
    out = _tc2(A2, h[:N2], SE2, W_nbr2[:1500], W_nbr2[1500:], W_self2,
               b2[None])
    return out


# speedup vs baseline: 7.3395x; 1.0545x over previous
import functools

import jax
import jax.numpy as jnp
from jax import lax
from jax.experimental import pallas as pl
from jax.experimental.pallas import tpu as pltpu
from jax.experimental.pallas import tpu_sc as plsc

NC, NS, LANES = 2, 16, 16
NW = NC * NS

N1, N2 = 2500, 1000
E1, E2 = 160000, 40000
G1, G2 = E1 // 16, E2 // 16
DE = 16

A1_ROWS, A2_ROWS = 2560, 1024
A1_SLOT_ROWS = 256
A2_SLOT_ROWS = 512
REG1 = A1_SLOT_ROWS * N1
REG2 = A2_SLOT_ROWS * N2
ASH = REG1 + 2560
ZSTRIPE = ASH // 80
ECH = 2000
EPT1 = E1 // NS
E2_PAD = 64000
EPT2 = E2_PAD // NS
DST_PAD_VAL = 800000
SE1_GCAP = 157
SE2_GCAP = 79
SE1_ROWS = 2560
SE1_W = SE1_ROWS * DE
SE2_W = 1024 * DE
EA_OFF = 41024

_MESH = plsc.VectorSubcoreMesh(core_axis_name="c", subcore_axis_name="s")


@functools.partial(
    pl.kernel,
    out_type=(
        jax.ShapeDtypeStruct((A1_ROWS * N1,), jnp.float32),
        jax.ShapeDtypeStruct((A2_ROWS * N2,), jnp.float32),
    ),
    mesh=_MESH,
    scratch_types=[
        pltpu.VMEM((18080,), jnp.float32),
        pltpu.VMEM((ECH,), jnp.int32),
        pltpu.VMEM((ECH,), jnp.int32),
        pltpu.VMEM((ECH,), jnp.int32),
        pltpu.VMEM((ECH,), jnp.float32),
        pltpu.VMEM_SHARED((ASH,), jnp.float32),
    ],
    compiler_params=pltpu.CompilerParams(needs_layout_passes=False),
)
def _sc_adj(src1, dst1, src2, dst2, a1_out, a2_out,
            buf, idx_s, idx_d, idx_w, ones_v, ash):
    core = lax.axis_index("c")
    tid = lax.axis_index("s")
    zeros16 = jnp.zeros((16,), jnp.float32)
    ones16 = jnp.ones((16,), jnp.float32)

    def zero_buf(nwords):
        def body(i, _):
            for u in range(10):
                buf[pl.ds(i * 160 + u * 16, 16)] = zeros16
            return 0
        lax.fori_loop(0, nwords // 160, body, 0)

    zero_buf(8160)

    def fill_ones(i, _):
        ones_v[pl.ds(i * 16, 16)] = ones16
        return 0
    lax.fori_loop(0, ECH // 16, fill_ones, 0)

    def adj_round(src, dst, ept, ncols, region, slot, iw, out_ref):
        lo = slot * region
        ureg = jnp.uint32(region)
        for z in range(5):
            pltpu.sync_copy(buf.at[pl.ds(0, ZSTRIPE)],
                            ash.at[pl.ds((tid * 5 + z) * ZSTRIPE, ZSTRIPE)])
        plsc.subcore_barrier()

        def chunk_body(c, _):
            tbase = tid * ept + c * ECH
            pltpu.sync_copy(src.at[pl.ds(tbase, ECH)], idx_s)
            pltpu.sync_copy(dst.at[pl.ds(tbase, ECH)], idx_d)

            def vec_body(k, _):
                s = idx_s[pl.ds(k * 16, 16)]
                d = idx_d[pl.ds(k * 16, 16)]
                local = d * ncols + s - lo
                valid = plsc.bitcast(local, jnp.uint32) < ureg
                iw[pl.ds(k * 16, 16)] = jnp.where(valid, local, region + s)
                return 0
            lax.fori_loop(0, ECH // 16, vec_body, 0)
            pltpu.sync_copy(ones_v, ash.at[iw], add=True)
            return 0
        lax.fori_loop(0, ept // ECH, chunk_body, 0)
        plsc.subcore_barrier()
        ostripe = region // 16
        sub = region // 80
        for z in range(5):
            off = tid * ostripe + z * sub
            pltpu.sync_copy(ash.at[pl.ds(off, sub)],
                            buf.at[pl.ds(10040, sub)])
            pltpu.sync_copy(buf.at[pl.ds(10040, sub)],
                            out_ref.at[pl.ds(lo + off, sub)])
        plsc.subcore_barrier()

    def a1_round(r, _):
        adj_round(src1, dst1, EPT1, N1, REG1, r * NC + core, idx_w, a1_out)
        return 0
    lax.fori_loop(0, 5, a1_round, 0)
    adj_round(src2, dst2, EPT2, N2, REG2, core, idx_w, a2_out)


@functools.partial(
    pl.kernel,
    out_type=(
        jax.ShapeDtypeStruct((NW * SE1_W,), jnp.float32),
        jax.ShapeDtypeStruct((NW * SE2_W,), jnp.float32),
    ),
    mesh=_MESH,
    scratch_types=[
        pltpu.VMEM((81280,), jnp.float32),
        pltpu.VMEM((2512,), jnp.int32),
    ],
    compiler_params=pltpu.CompilerParams(needs_layout_passes=False),
)
def _sc_se(dst1, ea1, dst2, ea2, se1_out, se2_out, buf, idx_d):
    core = lax.axis_index("c")
    tid = lax.axis_index("s")
    wid = tid * NC + core
    zeros16 = jnp.zeros((16,), jnp.float32)
    lane = lax.iota(jnp.int32, 16)

    def zero_buf(nwords):
        def body(i, _):
            for u in range(10):
                buf[pl.ds(i * 160 + u * 16, 16)] = zeros16
            return 0
        lax.fori_loop(0, nwords // 160, body, 0)

    def se_phase(dst, ea, groups, gcap, nchunk, nrows, out_ref):
        g0 = (wid * groups) // NW
        g1 = ((wid + 1) * groups) // NW
        priv_w = nrows * DE
        zero_buf(priv_w)
        priv = buf.at[pl.ds(0, priv_w)]
        ea_stage = buf.at[pl.ds(EA_OFF, gcap * 256)]
        for c in range(nchunk):
            start = g0 + c * gcap
            gb = jnp.minimum(start, groups - gcap)
            pltpu.sync_copy(dst.at[pl.ds(gb * 16, gcap * 16)],
                            idx_d.at[pl.ds(0, gcap * 16)])
            pltpu.sync_copy(ea.at[pl.ds(gb * 256, gcap * 256)], ea_stage)

            def g_body(k, _):
                d = idx_d[pl.ds(k * 16, 16)]
                rowbase = d * DE
                g = gb + k
                valid = jnp.broadcast_to((g >= start) & (g < g1), (16,))
                ebase = k * 256 + lane * 16
                for cc in range(DE):
                    vals = plsc.load_gather(ea_stage, [ebase + cc])
                    plsc.addupdate_scatter(priv, [rowbase + cc], vals,
                                           mask=valid)
                return 0
            lax.fori_loop(0, gcap, g_body, 0)
        pltpu.sync_copy(priv, out_ref.at[pl.ds(wid * priv_w, priv_w)])

    se_phase(dst1, ea1, G1, SE1_GCAP, 2, SE1_ROWS, se1_out)
    se_phase(dst2, ea2, G2, SE2_GCAP, 1, 1024, se2_out)


_TC1_BLK = 512
_TC1_GRID = A1_ROWS // _TC1_BLK


def _bdot(a, b):
    return jnp.dot(a.astype(jnp.bfloat16), b.astype(jnp.bfloat16),
                   preferred_element_type=jnp.float32)


def _sered_body(p_ref, o_ref):
    w = pl.program_id(0)

    @pl.when(w == 0)
    def _():
        o_ref[...] = p_ref[...]

    @pl.when(w > 0)
    def _():
        o_ref[...] = o_ref[...] + p_ref[...]


def _make_sered(width):
    return pl.pallas_call(
        _sered_body,
        grid=(NW,),
        in_specs=[pl.BlockSpec((width,), lambda w: (w,))],
        out_specs=pl.BlockSpec((width,), lambda w: (0,)),
        out_shape=jax.ShapeDtypeStruct((width,), jnp.float32),
        compiler_params=pltpu.CompilerParams(
            dimension_semantics=("arbitrary",)),
    )


_sered1 = _make_sered(SE1_W)
_sered2 = _make_sered(SE2_W)


def _tc1a_body(a1, x_full, x_blk, wself, b1, sinv_out, inv_out, t0_out):
    A = a1[...]
    S = _bdot(A, x_full[...])
    cnt = jnp.sum(A, axis=1, keepdims=True)
    inv = 1.0 / jnp.maximum(cnt, 1.0)
    sinv_out[...] = S * inv
    inv_out[...] = jnp.broadcast_to(inv, inv_out.shape)
    t0_out[...] = _bdot(x_blk[...], wself[...]) + b1[...]


_tc1a = pl.pallas_call(
    _tc1a_body,
    grid=(_TC1_GRID,),
    in_specs=[
        pl.BlockSpec((_TC1_BLK, N1), lambda i: (i, 0)),
        pl.BlockSpec((N1, 256), lambda i: (0, 0)),
        pl.BlockSpec((_TC1_BLK, 256), lambda i: (i, 0)),
        pl.BlockSpec((256, 1500), lambda i: (0, 0)),
        pl.BlockSpec((1, 1500), lambda i: (0, 0)),
    ],
    out_specs=[
        pl.BlockSpec((_TC1_BLK, 256), lambda i: (i, 0)),
        pl.BlockSpec((_TC1_BLK, 128), lambda i: (i, 0)),
        pl.BlockSpec((_TC1_BLK, 1500), lambda i: (i, 0)),
    ],
    out_shape=[
        jax.ShapeDtypeStruct((A1_ROWS, 256), jnp.float32),
        jax.ShapeDtypeStruct((A1_ROWS, 128), jnp.float32),
        jax.ShapeDtypeStruct((A1_ROWS, 1500), jnp.float32),
    ],
    compiler_params=pltpu.CompilerParams(dimension_semantics=("parallel",)),
)


def _tc1b_body(sinv, invb, se1, t0, wn1x, wn1e, gsc, beta, h_out):
    inv = invb[...][:, :1]
    t = _bdot(sinv[...], wn1x[...])
    t = t + jnp.dot(se1[...] * inv, wn1e[...],
                    preferred_element_type=jnp.float32)
    t = t + t0[...]
    h_out[...] = jnp.maximum(t, 0.0) * gsc[...] + beta[...]


_tc1b = pl.pallas_call(
    _tc1b_body,
    grid=(_TC1_GRID,),
    in_specs=[
        pl.BlockSpec((_TC1_BLK, 256), lambda i: (i, 0)),
        pl.BlockSpec((_TC1_BLK, 128), lambda i: (i, 0)),
        pl.BlockSpec((_TC1_BLK, DE), lambda i: (i, 0)),
        pl.BlockSpec((_TC1_BLK, 1500), lambda i: (i, 0)),
        pl.BlockSpec((256, 1500), lambda i: (0, 0)),
        pl.BlockSpec((DE, 1500), lambda i: (0, 0)),
        pl.BlockSpec((1, 1500), lambda i: (0, 0)),
        pl.BlockSpec((1, 1500), lambda i: (0, 0)),
    ],
    out_specs=pl.BlockSpec((_TC1_BLK, 1500), lambda i: (i, 0)),
    out_shape=jax.ShapeDtypeStruct((A1_ROWS, 1500), jnp.float32),
    compiler_params=pltpu.CompilerParams(dimension_semantics=("parallel",)),
)


def _tc2_body(a2, h1k, se2, wn2h, wn2e, wself2, b2, out):
    A = a2[...]
    H = h1k[...]
    P = _bdot(H, wn2h[...])
    G = _bdot(A, P)
    cnt = jnp.sum(A, axis=1, keepdims=True)
    inv = 1.0 / jnp.maximum(cnt, 1.0)
    o = (G * inv
         + jnp.dot(se2[...] * inv, wn2e[...],
                   preferred_element_type=jnp.float32)
         + _bdot(H, wself2[...])
         + b2[...])
    m = jnp.max(o, axis=1, keepdims=True)
    e = jnp.exp(o - m)
    out[...] = (o - m) - jnp.log(jnp.sum(e, axis=1, keepdims=True))


_tc2 = pl.pallas_call(
    _tc2_body,
    out_shape=jax.ShapeDtypeStruct((N2, 128), jnp.float32),
)


def kernel(x, res_size1, edge_index1, edge_attr1, res_size2, edge_index2,
           edge_attr2, W_nbr1, W_self1, b1, gamma, beta, W_nbr2, W_self2, b2):
    src1 = edge_index1[0].astype(jnp.int32)
    dst1 = edge_index1[1].astype(jnp.int32)
    src2 = jnp.concatenate([edge_index2[0].astype(jnp.int32),
                            jnp.arange(E2_PAD - E2, dtype=jnp.int32) % N2])
    dst2 = jnp.pad(edge_index2[1].astype(jnp.int32), (0, E2_PAD - E2),
                   constant_values=DST_PAD_VAL)
    ea1 = edge_attr1.reshape(-1)
    ea2 = edge_attr2.reshape(-1)

    a1f, a2f = _sc_adj(src1, dst1, src2, dst2)
    se1f, se2f = _sc_se(dst1, ea1, dst2, ea2)
    A1 = a1f.reshape(A1_ROWS, N1)
    A2 = a2f.reshape(A2_ROWS, N2)[:N2]
    SE1 = _sered1(se1f).reshape(SE1_ROWS, DE)
    SE2 = _sered2(se2f).reshape(1024, DE)[:N2]

    x25 = jnp.pad(x[:N1], ((0, A1_ROWS - N1), (0, 0)))
    gscale = (gamma * (1.0 / jnp.sqrt(jnp.float32(1.0 + 1e-5))))[None]

    sinv, invb, t0 = _tc1a(A1, x25[:N1], x25, W_self1, b1[None])
    h = _tc1b(sinv, invb, SE1, t0, W_nbr1[:256], W_nbr1[256:], gscale,
              beta[None])
    out = _tc2(A2, h[:N2], SE2, W_nbr2[:1500], W_nbr2[1500:], W_self2,
               b2[None])
    return out


# speedup vs baseline: 9.2648x; 1.2623x over previous
import functools

import jax
import jax.numpy as jnp
from jax import lax
from jax.experimental import pallas as pl
from jax.experimental.pallas import tpu as pltpu
from jax.experimental.pallas import tpu_sc as plsc

NC, NS, LANES = 2, 16, 16
NW = NC * NS

N1, N2 = 2500, 1000
E1, E2 = 160000, 40000
G1, G2 = E1 // 16, E2 // 16
DE = 16

A1_ROWS, A2_ROWS = 2560, 1024
A1_SLOT_ROWS = 640
A2_SLOT_ROWS = 512
REG1 = A1_SLOT_ROWS * N1
REG2 = A2_SLOT_ROWS * N2
ASH = REG1 + 2560
ZSTRIPE = ASH // 80
ECH = 2000
EPT1 = E1 // NS
E2_PAD = 64000
EPT2 = E2_PAD // NS
DST_PAD_VAL = 800000
SE1_GCAP = 157
SE2_GCAP = 79
SE1_ROWS = 2560
SE1_W = SE1_ROWS * DE
SE2_W = 1024 * DE
EA_OFF = 41024

_MESH = plsc.VectorSubcoreMesh(core_axis_name="c", subcore_axis_name="s")


@functools.partial(
    pl.kernel,
    out_type=(
        jax.ShapeDtypeStruct((A1_ROWS * N1,), jnp.float32),
        jax.ShapeDtypeStruct((A2_ROWS * N2,), jnp.float32),
    ),
    mesh=_MESH,
    scratch_types=[
        pltpu.VMEM((20160,), jnp.float32),
        pltpu.VMEM((ECH,), jnp.int32),
        pltpu.VMEM((ECH,), jnp.int32),
        pltpu.VMEM((ECH,), jnp.int32),
        pltpu.VMEM((ECH,), jnp.float32),
        pltpu.VMEM_SHARED((ASH,), jnp.float32),
    ],
    compiler_params=pltpu.CompilerParams(needs_layout_passes=False),
)
def _sc_adj(src1, dst1, src2, dst2, a1_out, a2_out,
            buf, idx_s, idx_d, idx_w, ones_v, ash):
    core = lax.axis_index("c")
    tid = lax.axis_index("s")
    zeros16 = jnp.zeros((16,), jnp.float32)
    ones16 = jnp.ones((16,), jnp.float32)

    def zero_buf(nwords):
        def body(i, _):
            for u in range(10):
                buf[pl.ds(i * 160 + u * 16, 16)] = zeros16
            return 0
        lax.fori_loop(0, nwords // 160, body, 0)

    zero_buf(20160)

    def fill_ones(i, _):
        ones_v[pl.ds(i * 16, 16)] = ones16
        return 0
    lax.fori_loop(0, ECH // 16, fill_ones, 0)

    def adj_round(src, dst, ept, ncols, region, slot, iw, out_ref):
        lo = slot * region
        ureg = jnp.uint32(region)
        for z in range(5):
            pltpu.sync_copy(buf.at[pl.ds(0, ZSTRIPE)],
                            ash.at[pl.ds((tid * 5 + z) * ZSTRIPE, ZSTRIPE)])
        plsc.subcore_barrier()

        def chunk_body(c, _):
            tbase = tid * ept + c * ECH
            pltpu.sync_copy(src.at[pl.ds(tbase, ECH)], idx_s)
            pltpu.sync_copy(dst.at[pl.ds(tbase, ECH)], idx_d)

            def vec_body(k, _):
                s = idx_s[pl.ds(k * 16, 16)]
                d = idx_d[pl.ds(k * 16, 16)]
                local = d * ncols + s - lo
                valid = plsc.bitcast(local, jnp.uint32) < ureg
                iw[pl.ds(k * 16, 16)] = jnp.where(valid, local, region + s)
                return 0
            lax.fori_loop(0, ECH // 16, vec_body, 0)
            pltpu.sync_copy(ones_v, ash.at[iw], add=True)
            return 0
        lax.fori_loop(0, ept // ECH, chunk_body, 0)
        plsc.subcore_barrier()
        ostripe = region // 16
        sub = region // 80
        for z in range(5):
            off = tid * ostripe + z * sub
            pltpu.sync_copy(ash.at[pl.ds(off, sub)],
                            buf.at[pl.ds(0, sub)])
            pltpu.sync_copy(buf.at[pl.ds(0, sub)],
                            out_ref.at[pl.ds(lo + off, sub)])
        zero_buf(20160)
        plsc.subcore_barrier()

    def a1_round(r, _):
        adj_round(src1, dst1, EPT1, N1, REG1, r * NC + core, idx_w, a1_out)
        return 0
    lax.fori_loop(0, 2, a1_round, 0)
    adj_round(src2, dst2, EPT2, N2, REG2, core, idx_w, a2_out)


@functools.partial(
    pl.kernel,
    out_type=(
        jax.ShapeDtypeStruct((NW * SE1_W,), jnp.float32),
        jax.ShapeDtypeStruct((NW * SE2_W,), jnp.float32),
    ),
    mesh=_MESH,
    scratch_types=[
        pltpu.VMEM((81280,), jnp.float32),
        pltpu.VMEM((2512,), jnp.int32),
    ],
    compiler_params=pltpu.CompilerParams(needs_layout_passes=False),
)
def _sc_se(dst1, ea1, dst2, ea2, se1_out, se2_out, buf, idx_d):
    core = lax.axis_index("c")
    tid = lax.axis_index("s")
    wid = tid * NC + core
    zeros16 = jnp.zeros((16,), jnp.float32)
    lane = lax.iota(jnp.int32, 16)

    def zero_buf(nwords):
        def body(i, _):
            for u in range(10):
                buf[pl.ds(i * 160 + u * 16, 16)] = zeros16
            return 0
        lax.fori_loop(0, nwords // 160, body, 0)

    def se_phase(dst, ea, groups, gcap, nchunk, nrows, out_ref):
        g0 = (wid * groups) // NW
        g1 = ((wid + 1) * groups) // NW
        priv_w = nrows * DE
        zero_buf(priv_w)
        priv = buf.at[pl.ds(0, priv_w)]
        ea_stage = buf.at[pl.ds(EA_OFF, gcap * 256)]
        for c in range(nchunk):
            start = g0 + c * gcap
            gb = jnp.minimum(start, groups - gcap)
            pltpu.sync_copy(dst.at[pl.ds(gb * 16, gcap * 16)],
                            idx_d.at[pl.ds(0, gcap * 16)])
            pltpu.sync_copy(ea.at[pl.ds(gb * 256, gcap * 256)], ea_stage)

            def g_body(k, _):
                d = idx_d[pl.ds(k * 16, 16)]
                rowbase = d * DE
                g = gb + k
                valid = jnp.broadcast_to((g >= start) & (g < g1), (16,))
                ebase = k * 256 + lane * 16
                for cc in range(DE):
                    vals = plsc.load_gather(ea_stage, [ebase + cc])
                    plsc.addupdate_scatter(priv, [rowbase + cc], vals,
                                           mask=valid)
                return 0
            lax.fori_loop(0, gcap, g_body, 0)
        pltpu.sync_copy(priv, out_ref.at[pl.ds(wid * priv_w, priv_w)])

    se_phase(dst1, ea1, G1, SE1_GCAP, 2, SE1_ROWS, se1_out)
    se_phase(dst2, ea2, G2, SE2_GCAP, 1, 1024, se2_out)


_TC1_BLK = 512
_TC1_GRID = A1_ROWS // _TC1_BLK


def _bdot(a, b):
    return jnp.dot(a.astype(jnp.bfloat16), b.astype(jnp.bfloat16),
                   preferred_element_type=jnp.float32)


def _make_sered(width):
    def body(p_ref, o_ref):
        v = p_ref[...]
        acc = v[0:width]
        for w in range(1, NW):
            acc = acc + v[w * width:(w + 1) * width]
        o_ref[...] = acc

    return pl.pallas_call(
        body,
        out_shape=jax.ShapeDtypeStruct((width,), jnp.float32),
    )


_sered1 = _make_sered(SE1_W)
_sered2 = _make_sered(SE2_W)


def _tc1a_body(a1, x_full, x_blk, wself, b1, sinv_out, inv_out, t0_out):
    A = a1[...]
    S = _bdot(A, x_full[...])
    cnt = jnp.sum(A, axis=1, keepdims=True)
    inv = 1.0 / jnp.maximum(cnt, 1.0)
    sinv_out[...] = S * inv
    inv_out[...] = jnp.broadcast_to(inv, inv_out.shape)
    t0_out[...] = _bdot(x_blk[...], wself[...]) + b1[...]


_tc1a = pl.pallas_call(
    _tc1a_body,
    grid=(_TC1_GRID,),
    in_specs=[
        pl.BlockSpec((_TC1_BLK, N1), lambda i: (i, 0)),
        pl.BlockSpec((N1, 256), lambda i: (0, 0)),
        pl.BlockSpec((_TC1_BLK, 256), lambda i: (i, 0)),
        pl.BlockSpec((256, 1500), lambda i: (0, 0)),
        pl.BlockSpec((1, 1500), lambda i: (0, 0)),
    ],
    out_specs=[
        pl.BlockSpec((_TC1_BLK, 256), lambda i: (i, 0)),
        pl.BlockSpec((_TC1_BLK, 128), lambda i: (i, 0)),
        pl.BlockSpec((_TC1_BLK, 1500), lambda i: (i, 0)),
    ],
    out_shape=[
        jax.ShapeDtypeStruct((A1_ROWS, 256), jnp.float32),
        jax.ShapeDtypeStruct((A1_ROWS, 128), jnp.float32),
        jax.ShapeDtypeStruct((A1_ROWS, 1500), jnp.float32),
    ],
    compiler_params=pltpu.CompilerParams(dimension_semantics=("parallel",)),
)


def _tc1b_body(sinv, invb, se1, t0, wn1x, wn1e, gsc, beta, h_out):
    inv = invb[...][:, :1]
    t = _bdot(sinv[...], wn1x[...])
    t = t + jnp.dot(se1[...] * inv, wn1e[...],
                    preferred_element_type=jnp.float32)
    t = t + t0[...]
    h_out[...] = jnp.maximum(t, 0.0) * gsc[...] + beta[...]


_tc1b = pl.pallas_call(
    _tc1b_body,
    grid=(_TC1_GRID,),
    in_specs=[
        pl.BlockSpec((_TC1_BLK, 256), lambda i: (i, 0)),
        pl.BlockSpec((_TC1_BLK, 128), lambda i: (i, 0)),
        pl.BlockSpec((_TC1_BLK, DE), lambda i: (i, 0)),
        pl.BlockSpec((_TC1_BLK, 1500), lambda i: (i, 0)),
        pl.BlockSpec((256, 1500), lambda i: (0, 0)),
        pl.BlockSpec((DE, 1500), lambda i: (0, 0)),
        pl.BlockSpec((1, 1500), lambda i: (0, 0)),
        pl.BlockSpec((1, 1500), lambda i: (0, 0)),
    ],
    out_specs=pl.BlockSpec((_TC1_BLK, 1500), lambda i: (i, 0)),
    out_shape=jax.ShapeDtypeStruct((A1_ROWS, 1500), jnp.float32),
    compiler_params=pltpu.CompilerParams(dimension_semantics=("parallel",)),
)


def _tc2_body(a2, h1k, se2, wn2h, wn2e, wself2, b2, out):
    A = a2[...]
    H = h1k[...]
    P = _bdot(H, wn2h[...])
    G = _bdot(A, P)
    cnt = jnp.sum(A, axis=1, keepdims=True)
    inv = 1.0 / jnp.maximum(cnt, 1.0)
    o = (G * inv
         + jnp.dot(se2[...] * inv, wn2e[...],
                   preferred_element_type=jnp.float32)
         + _bdot(H, wself2[...])
         + b2[...])
    m = jnp.max(o, axis=1, keepdims=True)
    e = jnp.exp(o - m)
    out[...] = (o - m) - jnp.log(jnp.sum(e, axis=1, keepdims=True))


_tc2 = pl.pallas_call(
    _tc2_body,
    out_shape=jax.ShapeDtypeStruct((N2, 128), jnp.float32),
)


def kernel(x, res_size1, edge_index1, edge_attr1, res_size2, edge_index2,
           edge_attr2, W_nbr1, W_self1, b1, gamma, beta, W_nbr2, W_self2, b2):
    src1 = edge_index1[0].astype(jnp.int32)
    dst1 = edge_index1[1].astype(jnp.int32)
    src2 = jnp.concatenate([edge_index2[0].astype(jnp.int32),
                            jnp.arange(E2_PAD - E2, dtype=jnp.int32) % N2])
    dst2 = jnp.pad(edge_index2[1].astype(jnp.int32), (0, E2_PAD - E2),
                   constant_values=DST_PAD_VAL)
    ea1 = edge_attr1.reshape(-1)
    ea2 = edge_attr2.reshape(-1)

    a1f, a2f = _sc_adj(src1, dst1, src2, dst2)
    se1f, se2f = _sc_se(dst1, ea1, dst2, ea2)
    A1 = a1f.reshape(A1_ROWS, N1)
    A2 = a2f.reshape(A2_ROWS, N2)[:N2]
    SE1 = _sered1(se1f).reshape(SE1_ROWS, DE)
    SE2 = _sered2(se2f).reshape(1024, DE)[:N2]

    x25 = jnp.pad(x[:N1], ((0, A1_ROWS - N1), (0, 0)))
    gscale = (gamma * (1.0 / jnp.sqrt(jnp.float32(1.0 + 1e-5))))[None]

    sinv, invb, t0 = _tc1a(A1, x25[:N1], x25, W_self1, b1[None])
    h = _tc1b(sinv, invb, SE1, t0, W_nbr1[:256], W_nbr1[256:], gscale,
              beta[None])
    out = _tc2(A2, h[:N2], SE2, W_nbr2[:1500], W_nbr2[1500:], W_self2,
               b2[None])
    return out


# speedup vs baseline: 9.6230x; 1.0387x over previous
import functools

import jax
import jax.numpy as jnp
from jax import lax
from jax.experimental import pallas as pl
from jax.experimental.pallas import tpu as pltpu
from jax.experimental.pallas import tpu_sc as plsc

NC, NS, LANES = 2, 16, 16
NW = NC * NS

N1, N2 = 2500, 1000
E1, E2 = 160000, 40000
G1, G2 = E1 // 16, E2 // 16
DE = 16

A1_ROWS, A2_ROWS = 2560, 1024
A1_SLOT_ROWS = 640
A2_SLOT_ROWS = 512
REG1 = A1_SLOT_ROWS * N1
REG2 = A2_SLOT_ROWS * N2
ASH = REG1 + 2560
ZSTRIPE = ASH // 80
ECH = 2000
EPT1 = E1 // NS
E2_PAD = 64000
EPT2 = E2_PAD // NS
DST_PAD_VAL = 800000
SE1_GCAP = 157
SE2_GCAP = 79
SE1_ROWS = 2560
SE1_W = SE1_ROWS * DE
SE2_W = 1024 * DE
EA_OFF = 41024

_MESH = plsc.VectorSubcoreMesh(core_axis_name="c", subcore_axis_name="s")


@functools.partial(
    pl.kernel,
    out_type=(
        jax.ShapeDtypeStruct((A1_ROWS * N1,), jnp.float32),
        jax.ShapeDtypeStruct((A2_ROWS * N2,), jnp.float32),
    ),
    mesh=_MESH,
    scratch_types=[
        pltpu.VMEM((20160,), jnp.float32),
        pltpu.VMEM((ECH,), jnp.int32),
        pltpu.VMEM((ECH,), jnp.int32),
        pltpu.VMEM((ECH,), jnp.int32),
        pltpu.VMEM((ECH,), jnp.float32),
        pltpu.VMEM_SHARED((ASH,), jnp.float32),
    ],
    compiler_params=pltpu.CompilerParams(needs_layout_passes=False),
)
def _sc_adj(src1, dst1, src2, dst2, a1_out, a2_out,
            buf, idx_s, idx_d, idx_w, ones_v, ash):
    core = lax.axis_index("c")
    tid = lax.axis_index("s")
    zeros16 = jnp.zeros((16,), jnp.float32)
    ones16 = jnp.ones((16,), jnp.float32)

    def zero_buf(nwords):
        def body(i, _):
            for u in range(10):
                buf[pl.ds(i * 160 + u * 16, 16)] = zeros16
            return 0
        lax.fori_loop(0, nwords // 160, body, 0)

    zero_buf(20160)

    def fill_ones(i, _):
        ones_v[pl.ds(i * 16, 16)] = ones16
        return 0
    lax.fori_loop(0, ECH // 16, fill_ones, 0)

    def adj_round(src, dst, ept, ncols, region, slot, iw, out_ref):
        lo = slot * region
        ureg = jnp.uint32(region)
        for z in range(5):
            pltpu.sync_copy(buf.at[pl.ds(0, ZSTRIPE)],
                            ash.at[pl.ds((tid * 5 + z) * ZSTRIPE, ZSTRIPE)])
        plsc.subcore_barrier()

        def chunk_body(c, _):
            tbase = tid * ept + c * ECH
            pltpu.sync_copy(src.at[pl.ds(tbase, ECH)], idx_s)
            pltpu.sync_copy(dst.at[pl.ds(tbase, ECH)], idx_d)

            def vec_body(k, _):
                s = idx_s[pl.ds(k * 16, 16)]
                d = idx_d[pl.ds(k * 16, 16)]
                local = d * ncols + s - lo
                valid = plsc.bitcast(local, jnp.uint32) < ureg
                iw[pl.ds(k * 16, 16)] = jnp.where(valid, local, region + s)
                return 0
            lax.fori_loop(0, ECH // 16, vec_body, 0)
            pltpu.sync_copy(ones_v, ash.at[iw], add=True)
            return 0
        lax.fori_loop(0, ept // ECH, chunk_body, 0)
        plsc.subcore_barrier()
        ostripe = region // 16
        sub = region // 80
        for z in range(5):
            off = tid * ostripe + z * sub
            pltpu.sync_copy(ash.at[pl.ds(off, sub)],
                            buf.at[pl.ds(0, sub)])
            pltpu.sync_copy(buf.at[pl.ds(0, sub)],
                            out_ref.at[pl.ds(lo + off, sub)])
        zero_buf(20160)
        plsc.subcore_barrier()

    def a1_round(r, _):
        adj_round(src1, dst1, EPT1, N1, REG1, r * NC + core, idx_w, a1_out)
        return 0
    lax.fori_loop(0, 2, a1_round, 0)
    adj_round(src2, dst2, EPT2, N2, REG2, core, idx_w, a2_out)


@functools.partial(
    pl.kernel,
    out_type=(
        jax.ShapeDtypeStruct((NW * SE1_W,), jnp.float32),
        jax.ShapeDtypeStruct((NW * SE2_W,), jnp.float32),
    ),
    mesh=_MESH,
    scratch_types=[
        pltpu.VMEM((81280,), jnp.float32),
        pltpu.VMEM((2512,), jnp.int32),
    ],
    compiler_params=pltpu.CompilerParams(needs_layout_passes=False),
)
def _sc_se(dst1, ea1, dst2, ea2, se1_out, se2_out, buf, idx_d):
    core = lax.axis_index("c")
    tid = lax.axis_index("s")
    wid = tid * NC + core
    zeros16 = jnp.zeros((16,), jnp.float32)
    lane = lax.iota(jnp.int32, 16)

    def zero_buf(nwords):
        def body(i, _):
            for u in range(10):
                buf[pl.ds(i * 160 + u * 16, 16)] = zeros16
            return 0
        lax.fori_loop(0, nwords // 160, body, 0)

    def se_phase(dst, ea, groups, gcap, nchunk, nrows, out_ref):
        g0 = (wid * groups) // NW
        g1 = ((wid + 1) * groups) // NW
        priv_w = nrows * DE
        zero_buf(priv_w)
        priv = buf.at[pl.ds(0, priv_w)]
        ea_stage = buf.at[pl.ds(EA_OFF, gcap * 256)]
        for c in range(nchunk):
            start = g0 + c * gcap
            gb = jnp.minimum(start, groups - gcap)
            pltpu.sync_copy(dst.at[pl.ds(gb * 16, gcap * 16)],
                            idx_d.at[pl.ds(0, gcap * 16)])
            pltpu.sync_copy(ea.at[pl.ds(gb * 256, gcap * 256)], ea_stage)

            def g_body(k, _):
                d = idx_d[pl.ds(k * 16, 16)]
                rowbase = d * DE
                g = gb + k
                valid = jnp.broadcast_to((g >= start) & (g < g1), (16,))
                ebase = k * 256 + lane * 16
                for cc in range(DE):
                    vals = plsc.load_gather(ea_stage, [ebase + cc])
                    plsc.addupdate_scatter(priv, [rowbase + cc], vals,
                                           mask=valid)
                return 0
            lax.fori_loop(0, gcap, g_body, 0)
        pltpu.sync_copy(priv, out_ref.at[pl.ds(wid * priv_w, priv_w)])

    se_phase(dst1, ea1, G1, SE1_GCAP, 2, SE1_ROWS, se1_out)
    se_phase(dst2, ea2, G2, SE2_GCAP, 1, 1024, se2_out)


_TC1_BLK = 512
_TC1_GRID = A1_ROWS // _TC1_BLK


def _bdot(a, b):
    return jnp.dot(a.astype(jnp.bfloat16), b.astype(jnp.bfloat16),
                   preferred_element_type=jnp.float32)


def _make_sered(width):
    def body(p_ref, o_ref):
        v = p_ref[...]
        acc = v[0:width]
        for w in range(1, NW):
            acc = acc + v[w * width:(w + 1) * width]
        o_ref[...] = acc

    return pl.pallas_call(
        body,
        out_shape=jax.ShapeDtypeStruct((width,), jnp.float32),
    )


_sered1 = _make_sered(SE1_W)
_sered2 = _make_sered(SE2_W)


def _tc1a_body(a1, x_full, x_blk, wself, b1, sinv_out, inv_out, t0_out):
    A = a1[...]
    S = _bdot(A, x_full[...])
    cnt = jnp.sum(A, axis=1, keepdims=True)
    inv = 1.0 / jnp.maximum(cnt, 1.0)
    sinv_out[...] = S * inv
    inv_out[...] = jnp.broadcast_to(inv, inv_out.shape)
    t0_out[...] = _bdot(x_blk[...], wself[...]) + b1[...]


_tc1a = pl.pallas_call(
    _tc1a_body,
    grid=(_TC1_GRID,),
    in_specs=[
        pl.BlockSpec((_TC1_BLK, N1), lambda i: (i, 0)),
        pl.BlockSpec((N1, 256), lambda i: (0, 0)),
        pl.BlockSpec((_TC1_BLK, 256), lambda i: (i, 0)),
        pl.BlockSpec((256, 1500), lambda i: (0, 0)),
        pl.BlockSpec((1, 1500), lambda i: (0, 0)),
    ],
    out_specs=[
        pl.BlockSpec((_TC1_BLK, 256), lambda i: (i, 0)),
        pl.BlockSpec((_TC1_BLK, 128), lambda i: (i, 0)),
        pl.BlockSpec((_TC1_BLK, 1500), lambda i: (i, 0)),
    ],
    out_shape=[
        jax.ShapeDtypeStruct((A1_ROWS, 256), jnp.float32),
        jax.ShapeDtypeStruct((A1_ROWS, 128), jnp.float32),
        jax.ShapeDtypeStruct((A1_ROWS, 1500), jnp.float32),
    ],
    compiler_params=pltpu.CompilerParams(dimension_semantics=("parallel",)),
)


def _tc1b_body(sinv, invb, se1, t0, wn1x, wn1e, gsc, beta, h_out):
    inv = invb[...][:, :1]
    t = _bdot(sinv[...], wn1x[...])
    t = t + jnp.dot(se1[...] * inv, wn1e[...],
                    preferred_element_type=jnp.float32)
    t = t + t0[...]
    h_out[...] = (jnp.maximum(t, 0.0) * gsc[...] + beta[...]).astype(
        jnp.bfloat16)


_tc1b = pl.pallas_call(
    _tc1b_body,
    grid=(_TC1_GRID,),
    in_specs=[
        pl.BlockSpec((_TC1_BLK, 256), lambda i: (i, 0)),
        pl.BlockSpec((_TC1_BLK, 128), lambda i: (i, 0)),
        pl.BlockSpec((_TC1_BLK, DE), lambda i: (i, 0)),
        pl.BlockSpec((_TC1_BLK, 1500), lambda i: (i, 0)),
        pl.BlockSpec((256, 1500), lambda i: (0, 0)),
        pl.BlockSpec((DE, 1500), lambda i: (0, 0)),
        pl.BlockSpec((1, 1500), lambda i: (0, 0)),
        pl.BlockSpec((1, 1500), lambda i: (0, 0)),
    ],
    out_specs=pl.BlockSpec((_TC1_BLK, 1500), lambda i: (i, 0)),
    out_shape=jax.ShapeDtypeStruct((A1_ROWS, 1500), jnp.bfloat16),
    compiler_params=pltpu.CompilerParams(dimension_semantics=("parallel",)),
)


def _tc2_body(a2, h1k, se2, wn2h, wn2e, wself2, b2, out):
    A = a2[...]
    H = h1k[...]
    P = _bdot(H, wn2h[...])
    G = _bdot(A, P)
    cnt = jnp.sum(A, axis=1, keepdims=True)
    inv = 1.0 / jnp.maximum(cnt, 1.0)
    o = (G * inv
         + jnp.dot(se2[...] * inv, wn2e[...],
                   preferred_element_type=jnp.float32)
         + _bdot(H, wself2[...])
         + b2[...])
    m = jnp.max(o, axis=1, keepdims=True)
    e = jnp.exp(o - m)
    out[...] = (o - m) - jnp.log(jnp.sum(e, axis=1, keepdims=True))


_tc2 = pl.pallas_call(
    _tc2_body,
    grid=(1,),
    in_specs=[
        pl.BlockSpec((N2, N2), lambda i: (0, 0)),
        pl.BlockSpec((N2, 1500), lambda i: (0, 0)),
        pl.BlockSpec((N2, DE), lambda i: (0, 0)),
        pl.BlockSpec((1500, 128), lambda i: (0, 0)),
        pl.BlockSpec((DE, 128), lambda i: (0, 0)),
        pl.BlockSpec((1500, 128), lambda i: (0, 0)),
        pl.BlockSpec((1, 128), lambda i: (0, 0)),
    ],
    out_specs=pl.BlockSpec((N2, 128), lambda i: (0, 0)),
    out_shape=jax.ShapeDtypeStruct((N2, 128), jnp.float32),
)


def kernel(x, res_size1, edge_index1, edge_attr1, res_size2, edge_index2,
           edge_attr2, W_nbr1, W_self1, b1, gamma, beta, W_nbr2, W_self2, b2):
    src1 = edge_index1[0].astype(jnp.int32)
    dst1 = edge_index1[1].astype(jnp.int32)
    src2 = jnp.concatenate([edge_index2[0].astype(jnp.int32),
                            jnp.arange(E2_PAD - E2, dtype=jnp.int32) % N2])
    dst2 = jnp.pad(edge_index2[1].astype(jnp.int32), (0, E2_PAD - E2),
                   constant_values=DST_PAD_VAL)
    ea1 = edge_attr1.reshape(-1)
    ea2 = edge_attr2.reshape(-1)

    a1f, a2f = _sc_adj(src1, dst1, src2, dst2)
    se1f, se2f = _sc_se(dst1, ea1, dst2, ea2)
    A1 = a1f.reshape(A1_ROWS, N1)
    A2 = a2f.reshape(A2_ROWS, N2)
    SE1 = _sered1(se1f).reshape(SE1_ROWS, DE)
    SE2 = _sered2(se2f).reshape(1024, DE)

    x25 = jnp.pad(x[:N1], ((0, A1_ROWS - N1), (0, 0)))
    gscale = (gamma * (1.0 / jnp.sqrt(jnp.float32(1.0 + 1e-5))))[None]

    sinv, invb, t0 = _tc1a(A1, x25[:N1], x25, W_self1, b1[None])
    h = _tc1b(sinv, invb, SE1, t0, W_nbr1[:256], W_nbr1[256:], gscale,
              beta[None])
    out = _tc2(A2, h, SE2, W_nbr2[:1500], W_nbr2[1500:], W_self2,
               b2[None])
    return out


# speedup vs baseline: 10.8124x; 1.1236x over previous
import functools

import jax
import jax.numpy as jnp
from jax import lax
from jax.experimental import pallas as pl
from jax.experimental.pallas import tpu as pltpu
from jax.experimental.pallas import tpu_sc as plsc

NC, NS, LANES = 2, 16, 16
NW = NC * NS

N1, N2 = 2500, 1000
E1, E2 = 160000, 40000
G1, G2 = E1 // 16, E2 // 16
DE = 16

A1_ROWS, A2_ROWS = 2560, 1024
A1_SLOT_ROWS = 640
A2_SLOT_ROWS = 512
REG1 = A1_SLOT_ROWS * N1
REG2 = A2_SLOT_ROWS * N2
ASH = REG1 + 2560
ZSTRIPE = ASH // 80
ECH = 2000
EPT1 = E1 // NS
E2_PAD = 64000
EPT2 = E2_PAD // NS
DST_PAD_VAL = 800000
SE1_GCAP = 157
SE2_GCAP = 79
SE1_ROWS = 2560
SE1_W = SE1_ROWS * DE
SE2_W = 1024 * DE
EA_OFF = 41024

_MESH = plsc.VectorSubcoreMesh(core_axis_name="c", subcore_axis_name="s")


@functools.partial(
    pl.kernel,
    out_type=(
        jax.ShapeDtypeStruct((A1_ROWS * N1,), jnp.float32),
        jax.ShapeDtypeStruct((A2_ROWS * N2,), jnp.float32),
    ),
    mesh=_MESH,
    scratch_types=[
        pltpu.VMEM((20160,), jnp.float32),
        pltpu.VMEM((ECH,), jnp.int32),
        pltpu.VMEM((ECH,), jnp.int32),
        pltpu.VMEM((ECH,), jnp.int32),
        pltpu.VMEM((ECH,), jnp.float32),
        pltpu.VMEM_SHARED((ASH,), jnp.float32),
    ],
    compiler_params=pltpu.CompilerParams(needs_layout_passes=False),
)
def _sc_adj(src1, dst1, src2, dst2, a1_out, a2_out,
            buf, idx_s, idx_d, idx_w, ones_v, ash):
    core = lax.axis_index("c")
    tid = lax.axis_index("s")
    zeros16 = jnp.zeros((16,), jnp.float32)
    ones16 = jnp.ones((16,), jnp.float32)

    def zero_buf(nwords):
        def body(i, _):
            for u in range(10):
                buf[pl.ds(i * 160 + u * 16, 16)] = zeros16
            return 0
        lax.fori_loop(0, nwords // 160, body, 0)

    zero_buf(20160)

    def fill_ones(i, _):
        ones_v[pl.ds(i * 16, 16)] = ones16
        return 0
    lax.fori_loop(0, ECH // 16, fill_ones, 0)

    def adj_round(src, dst, ept, ncols, region, slot, iw, out_ref):
        lo = slot * region
        ureg = jnp.uint32(region)
        for z in range(5):
            pltpu.sync_copy(buf.at[pl.ds(0, ZSTRIPE)],
                            ash.at[pl.ds((tid * 5 + z) * ZSTRIPE, ZSTRIPE)])
        plsc.subcore_barrier()

        def chunk_body(c, _):
            tbase = tid * ept + c * ECH
            pltpu.sync_copy(src.at[pl.ds(tbase, ECH)], idx_s)
            pltpu.sync_copy(dst.at[pl.ds(tbase, ECH)], idx_d)

            def vec_body(k, _):
                s = idx_s[pl.ds(k * 16, 16)]
                d = idx_d[pl.ds(k * 16, 16)]
                local = d * ncols + s - lo
                valid = plsc.bitcast(local, jnp.uint32) < ureg
                iw[pl.ds(k * 16, 16)] = jnp.where(valid, local, region + s)
                return 0
            lax.fori_loop(0, ECH // 16, vec_body, 0)
            pltpu.sync_copy(ones_v, ash.at[iw], add=True)
            return 0
        lax.fori_loop(0, ept // ECH, chunk_body, 0)
        plsc.subcore_barrier()
        ostripe = region // 16
        sub = region // 80
        for z in range(5):
            off = tid * ostripe + z * sub
            pltpu.sync_copy(ash.at[pl.ds(off, sub)],
                            buf.at[pl.ds(0, sub)])
            pltpu.sync_copy(buf.at[pl.ds(0, sub)],
                            out_ref.at[pl.ds(lo + off, sub)])
        zero_buf(20160)
        plsc.subcore_barrier()

    def a1_round(r, _):
        adj_round(src1, dst1, EPT1, N1, REG1, r * NC + core, idx_w, a1_out)
        return 0
    lax.fori_loop(0, 2, a1_round, 0)
    adj_round(src2, dst2, EPT2, N2, REG2, core, idx_w, a2_out)


@functools.partial(
    pl.kernel,
    out_type=(
        jax.ShapeDtypeStruct((NW * SE1_W,), jnp.float32),
        jax.ShapeDtypeStruct((NW * SE2_W,), jnp.float32),
    ),
    mesh=_MESH,
    scratch_types=[
        pltpu.VMEM((81280,), jnp.float32),
        pltpu.VMEM((2512,), jnp.int32),
    ],
    compiler_params=pltpu.CompilerParams(needs_layout_passes=False),
)
def _sc_se(dst1, ea1, dst2, ea2, se1_out, se2_out, buf, idx_d):
    core = lax.axis_index("c")
    tid = lax.axis_index("s")
    wid = tid * NC + core
    zeros16 = jnp.zeros((16,), jnp.float32)
    lane = lax.iota(jnp.int32, 16)

    def zero_buf(nwords):
        def body(i, _):
            for u in range(10):
                buf[pl.ds(i * 160 + u * 16, 16)] = zeros16
            return 0
        lax.fori_loop(0, nwords // 160, body, 0)

    def se_phase(dst, ea, groups, gcap, nchunk, nrows, out_ref):
        g0 = (wid * groups) // NW
        g1 = ((wid + 1) * groups) // NW
        priv_w = nrows * DE
        zero_buf(priv_w)
        priv = buf.at[pl.ds(0, priv_w)]
        ea_stage = buf.at[pl.ds(EA_OFF, gcap * 256)]
        for c in range(nchunk):
            start = g0 + c * gcap
            gb = jnp.minimum(start, groups - gcap)
            pltpu.sync_copy(dst.at[pl.ds(gb * 16, gcap * 16)],
                            idx_d.at[pl.ds(0, gcap * 16)])
            pltpu.sync_copy(ea.at[pl.ds(gb * 256, gcap * 256)], ea_stage)

            def g_body(k, _):
                d = idx_d[pl.ds(k * 16, 16)]
                g = gb + k
                scale = jnp.where((g >= start) & (g < g1),
                                  jnp.float32(1.0), jnp.float32(0.0))
                dm = jnp.minimum(d, nrows - 1)
                for e in range(16):
                    d_e = jnp.sum(jnp.where(lane == e, dm, 0))
                    vals = ea_stage[pl.ds((k * 16 + e) * 16, 16)]
                    plsc.addupdate(priv.at[pl.ds(d_e * DE, DE)],
                                   vals * scale)
                return 0
            lax.fori_loop(0, gcap, g_body, 0)
        pltpu.sync_copy(priv, out_ref.at[pl.ds(wid * priv_w, priv_w)])

    se_phase(dst1, ea1, G1, SE1_GCAP, 2, SE1_ROWS, se1_out)
    se_phase(dst2, ea2, G2, SE2_GCAP, 1, 1024, se2_out)


_TC1_BLK = 512
_TC1_GRID = A1_ROWS // _TC1_BLK


def _bdot(a, b):
    return jnp.dot(a.astype(jnp.bfloat16), b.astype(jnp.bfloat16),
                   preferred_element_type=jnp.float32)


def _make_sered(width):
    def body(p_ref, o_ref):
        v = p_ref[...]
        acc = v[0:width]
        for w in range(1, NW):
            acc = acc + v[w * width:(w + 1) * width]
        o_ref[...] = acc

    return pl.pallas_call(
        body,
        out_shape=jax.ShapeDtypeStruct((width,), jnp.float32),
    )


_sered1 = _make_sered(SE1_W)
_sered2 = _make_sered(SE2_W)


def _tc1a_body(a1, x_full, x_blk, wself, b1, sinv_out, inv_out, t0_out):
    A = a1[...]
    S = _bdot(A, x_full[...])
    cnt = jnp.sum(A, axis=1, keepdims=True)
    inv = 1.0 / jnp.maximum(cnt, 1.0)
    sinv_out[...] = S * inv
    inv_out[...] = jnp.broadcast_to(inv, inv_out.shape)
    t0_out[...] = _bdot(x_blk[...], wself[...]) + b1[...]


_tc1a = pl.pallas_call(
    _tc1a_body,
    grid=(_TC1_GRID,),
    in_specs=[
        pl.BlockSpec((_TC1_BLK, N1), lambda i: (i, 0)),
        pl.BlockSpec((N1, 256), lambda i: (0, 0)),
        pl.BlockSpec((_TC1_BLK, 256), lambda i: (i, 0)),
        pl.BlockSpec((256, 1500), lambda i: (0, 0)),
        pl.BlockSpec((1, 1500), lambda i: (0, 0)),
    ],
    out_specs=[
        pl.BlockSpec((_TC1_BLK, 256), lambda i: (i, 0)),
        pl.BlockSpec((_TC1_BLK, 128), lambda i: (i, 0)),
        pl.BlockSpec((_TC1_BLK, 1500), lambda i: (i, 0)),
    ],
    out_shape=[
        jax.ShapeDtypeStruct((A1_ROWS, 256), jnp.float32),
        jax.ShapeDtypeStruct((A1_ROWS, 128), jnp.float32),
        jax.ShapeDtypeStruct((A1_ROWS, 1500), jnp.float32),
    ],
    compiler_params=pltpu.CompilerParams(dimension_semantics=("parallel",)),
)


def _tc1b_body(sinv, invb, se1, t0, wn1x, wn1e, gsc, beta, h_out):
    inv = invb[...][:, :1]
    t = _bdot(sinv[...], wn1x[...])
    t = t + jnp.dot(se1[...] * inv, wn1e[...],
                    preferred_element_type=jnp.float32)
    t = t + t0[...]
    h_out[...] = (jnp.maximum(t, 0.0) * gsc[...] + beta[...]).astype(
        jnp.bfloat16)


_tc1b = pl.pallas_call(
    _tc1b_body,
    grid=(_TC1_GRID,),
    in_specs=[
        pl.BlockSpec((_TC1_BLK, 256), lambda i: (i, 0)),
        pl.BlockSpec((_TC1_BLK, 128), lambda i: (i, 0)),
        pl.BlockSpec((_TC1_BLK, DE), lambda i: (i, 0)),
        pl.BlockSpec((_TC1_BLK, 1500), lambda i: (i, 0)),
        pl.BlockSpec((256, 1500), lambda i: (0, 0)),
        pl.BlockSpec((DE, 1500), lambda i: (0, 0)),
        pl.BlockSpec((1, 1500), lambda i: (0, 0)),
        pl.BlockSpec((1, 1500), lambda i: (0, 0)),
    ],
    out_specs=pl.BlockSpec((_TC1_BLK, 1500), lambda i: (i, 0)),
    out_shape=jax.ShapeDtypeStruct((A1_ROWS, 1500), jnp.bfloat16),
    compiler_params=pltpu.CompilerParams(dimension_semantics=("parallel",)),
)


def _tc2_body(a2, h1k, se2, wn2h, wn2e, wself2, b2, out):
    A = a2[...]
    H = h1k[...]
    P = _bdot(H, wn2h[...])
    G = _bdot(A, P)
    cnt = jnp.sum(A, axis=1, keepdims=True)
    inv = 1.0 / jnp.maximum(cnt, 1.0)
    o = (G * inv
         + jnp.dot(se2[...] * inv, wn2e[...],
                   preferred_element_type=jnp.float32)
         + _bdot(H, wself2[...])
         + b2[...])
    m = jnp.max(o, axis=1, keepdims=True)
    e = jnp.exp(o - m)
    out[...] = (o - m) - jnp.log(jnp.sum(e, axis=1, keepdims=True))


_tc2 = pl.pallas_call(
    _tc2_body,
    grid=(1,),
    in_specs=[
        pl.BlockSpec((N2, N2), lambda i: (0, 0)),
        pl.BlockSpec((N2, 1500), lambda i: (0, 0)),
        pl.BlockSpec((N2, DE), lambda i: (0, 0)),
        pl.BlockSpec((1500, 128), lambda i: (0, 0)),
        pl.BlockSpec((DE, 128), lambda i: (0, 0)),
        pl.BlockSpec((1500, 128), lambda i: (0, 0)),
        pl.BlockSpec((1, 128), lambda i: (0, 0)),
    ],
    out_specs=pl.BlockSpec((N2, 128), lambda i: (0, 0)),
    out_shape=jax.ShapeDtypeStruct((N2, 128), jnp.float32),
)


def kernel(x, res_size1, edge_index1, edge_attr1, res_size2, edge_index2,
           edge_attr2, W_nbr1, W_self1, b1, gamma, beta, W_nbr2, W_self2, b2):
    src1 = edge_index1[0].astype(jnp.int32)
    dst1 = edge_index1[1].astype(jnp.int32)
    src2 = jnp.concatenate([edge_index2[0].astype(jnp.int32),
                            jnp.arange(E2_PAD - E2, dtype=jnp.int32) % N2])
    dst2 = jnp.pad(edge_index2[1].astype(jnp.int32), (0, E2_PAD - E2),
                   constant_values=DST_PAD_VAL)
    ea1 = edge_attr1.reshape(-1)
    ea2 = edge_attr2.reshape(-1)

    a1f, a2f = _sc_adj(src1, dst1, src2, dst2)
    se1f, se2f = _sc_se(dst1, ea1, dst2, ea2)
    A1 = a1f.reshape(A1_ROWS, N1)
    A2 = a2f.reshape(A2_ROWS, N2)
    SE1 = _sered1(se1f).reshape(SE1_ROWS, DE)
    SE2 = _sered2(se2f).reshape(1024, DE)

    x25 = jnp.pad(x[:N1], ((0, A1_ROWS - N1), (0, 0)))
    gscale = (gamma * (1.0 / jnp.sqrt(jnp.float32(1.0 + 1e-5))))[None]

    sinv, invb, t0 = _tc1a(A1, x25[:N1], x25, W_self1, b1[None])
    h = _tc1b(sinv, invb, SE1, t0, W_nbr1[:256], W_nbr1[256:], gscale,
              beta[None])
    out = _tc2(A2, h, SE2, W_nbr2[:1500], W_nbr2[1500:], W_self2,
               b2[None])
    return out
